# Initial kernel scaffold; baseline (speedup 1.0000x reference)
import jax
import jax.numpy as jnp
from jax.experimental import pallas as pl


def kernel(x, edge_index, W1, al1, ar1, b1, W2, al2, ar2, b2):
    raise NotImplementedError("write your pallas kernel here")


# speedup vs baseline: 54.3864x; 54.3864x over previous
import dataclasses
import functools

import jax
import jax.numpy as jnp
from jax import lax
from jax.experimental import pallas as pl
from jax.experimental.pallas import tpu as pltpu
from jax.experimental.pallas import tpu_sc as plsc

N = 10000
E = 320000
F_IN = 128
H1, F1 = 8, 8
D1 = H1 * F1
H2, F2 = 1, 40
D2 = H2 * F2

TW1 = 80
TW2 = 48
ERW = 16

NTILES = 32
CHUNK = 80
EPT = E // NTILES
NCH = EPT // CHUNK
R0 = 624
RLAST = N - 15 * R0

def _mesh():
    return plsc.VectorSubcoreMesh(core_axis_name="c", subcore_axis_name="s")


def _sc_params():
    cp = pltpu.CompilerParams()
    fields = pltpu.CompilerParams.__dataclass_fields__
    if "needs_layout_passes" in fields:
        cp = dataclasses.replace(cp, needs_layout_passes=False)
    if "use_tc_tiling_on_sc" in fields:
        cp = dataclasses.replace(cp, use_tc_tiling_on_sc=False)
    return cp


def _mm_body(x_ref, w_ref, a_ref, b_ref, split):
    y = jnp.dot(x_ref[...], w_ref[...], preferred_element_type=jnp.float32)
    a_ref[...] = y[:, :split]
    b_ref[...] = y[:, split:]


def _tc_project(x, w, split, rows_per_blk=1000):
    n, k = x.shape
    m = w.shape[1]
    grid = (n // rows_per_blk,)
    return pl.pallas_call(
        functools.partial(_mm_body, split=split),
        grid=grid,
        in_specs=[
            pl.BlockSpec((rows_per_blk, k), lambda i: (i, 0)),
            pl.BlockSpec((k, m), lambda i: (0, 0)),
        ],
        out_specs=[
            pl.BlockSpec((rows_per_blk, split), lambda i: (i, 0)),
            pl.BlockSpec((rows_per_blk, m - split), lambda i: (i, 0)),
        ],
        out_shape=[
            jax.ShapeDtypeStruct((n, split), jnp.float32),
            jax.ShapeDtypeStruct((n, m - split), jnp.float32),
        ],
    )(x, w)


def _mid_body(p_ref, w_ref, b_ref, t2_ref, er_ref):
    a = p_ref[0] + p_ref[1]
    pieces = []
    for h in range(H1):
        d = a[:, D1 + h:D1 + h + 1]
        d = jnp.where(d != 0.0, d, 1.0)
        pieces.append(a[:, F1 * h:F1 * h + F1] / d)
    o = jnp.concatenate(pieces, axis=1)
    hb = jnp.maximum(o + b_ref[...], 0.0)
    y = jnp.dot(hb, w_ref[...], preferred_element_type=jnp.float32)
    t2_ref[...] = y[:, :TW2]
    er_ref[...] = y[:, TW2:]


def _tc_mid(p1, w, b1row, rows_per_blk=1000):
    grid = (N // rows_per_blk,)
    return pl.pallas_call(
        _mid_body,
        grid=grid,
        in_specs=[
            pl.BlockSpec((2, rows_per_blk, TW1), lambda i: (0, i, 0)),
            pl.BlockSpec(w.shape, lambda i: (0, 0)),
            pl.BlockSpec((1, D1), lambda i: (0, 0)),
        ],
        out_specs=[
            pl.BlockSpec((rows_per_blk, TW2), lambda i: (i, 0)),
            pl.BlockSpec((rows_per_blk, ERW), lambda i: (i, 0)),
        ],
        out_shape=[
            jax.ShapeDtypeStruct((N, TW2), jnp.float32),
            jax.ShapeDtypeStruct((N, ERW), jnp.float32),
        ],
    )(p1, w, b1row)


def _final_body(p_ref, b_ref, o_ref):
    a = p_ref[0] + p_ref[1]
    d = a[:, F2:F2 + 1]
    d = jnp.where(d != 0.0, d, 1.0)
    z = a[:, :F2] / d + b_ref[...]
    m = jnp.max(z, axis=1, keepdims=True)
    lse = m + jnp.log(jnp.sum(jnp.exp(z - m), axis=1, keepdims=True))
    o_ref[...] = z - lse


def _tc_final(p2, b2row, rows_per_blk=1000):
    grid = (N // rows_per_blk,)
    return pl.pallas_call(
        _final_body,
        grid=grid,
        in_specs=[
            pl.BlockSpec((2, rows_per_blk, TW2), lambda i: (0, i, 0)),
            pl.BlockSpec((1, F2), lambda i: (0, 0)),
        ],
        out_specs=pl.BlockSpec((rows_per_blk, F2), lambda i: (i, 0)),
        out_shape=jax.ShapeDtypeStruct((N, F2), jnp.float32),
    )(p2, b2row)


def _sc_layer1_body(t1_hbm, er_hbm, src_hbm, dst_hbm, z_hbm, out_hbm,
                    srcv, dstv, t1b, erb, msgb, acc, sem1, sem2):
    cid = lax.axis_index("c")
    sid = lax.axis_index("s")
    wid = cid * 16 + sid
    _rowcopy(z_hbm, acc, sid)
    pltpu.sync_copy(src_hbm.at[wid], srcv)
    pltpu.sync_copy(dst_hbm.at[wid], dstv)
    plsc.subcore_barrier()

    il = lax.iota(jnp.int32, 16)
    head_sel = il >> 3

    @pl.loop(0, NCH)
    def _chunk(j):
        g1 = pltpu.async_copy(t1_hbm.at[srcv.at[j]], t1b, sem1)
        g2 = pltpu.async_copy(er_hbm.at[dstv.at[j]], erb, sem2)
        g1.wait()
        g2.wait()

        @pl.loop(0, CHUNK)
        def _edge(e):
            elv = t1b[e, pl.ds(D1, 16)]
            erv = erb[e, :]
            ev = elv + erv
            ev = jnp.where(ev >= 0.0, ev, 0.2 * ev)
            sv = jnp.exp(ev)
            msgb[e, pl.ds(D1, 16)] = sv
            for q in range(4):
                hv = t1b[e, pl.ds(16 * q, 16)]
                col = D1 + 2 * q + head_sel
                sb = plsc.load_gather(
                    msgb, (jnp.full((16,), e, jnp.int32), col))
                msgb[e, pl.ds(16 * q, 16)] = hv * sb

        pltpu.sync_copy(msgb, acc.at[dstv.at[j]], add=True)

    plsc.subcore_barrier()
    _rowcopy(acc, out_hbm.at[cid], sid)


def _rowcopy(src, dst, sid):
    @pl.when(sid < 15)
    def _():
        st = pl.multiple_of(sid * R0, 8)
        pltpu.sync_copy(src.at[pl.ds(st, R0)], dst.at[pl.ds(st, R0)])

    @pl.when(sid == 15)
    def _():
        pltpu.sync_copy(src.at[pl.ds(15 * R0, RLAST)],
                        dst.at[pl.ds(15 * R0, RLAST)])


def _sc_layer1(t1, er1, src_r, dst_r, z80):
    k = pl.kernel(
        _sc_layer1_body,
        out_type=jax.ShapeDtypeStruct((2, N, TW1), jnp.float32),
        mesh=_mesh(),
        compiler_params=_sc_params(),
        scratch_types=[
            pltpu.VMEM((NCH, CHUNK), jnp.int32),
            pltpu.VMEM((NCH, CHUNK), jnp.int32),
            pltpu.VMEM((CHUNK, TW1), jnp.float32),
            pltpu.VMEM((CHUNK, ERW), jnp.float32),
            pltpu.VMEM((CHUNK, TW1), jnp.float32),
            pltpu.VMEM_SHARED((N, TW1), jnp.float32),
            pltpu.SemaphoreType.DMA,
            pltpu.SemaphoreType.DMA,
        ],
    )
    return k(t1, er1, src_r, dst_r, z80)


def _sc_layer2_body(t2_hbm, er_hbm, src_hbm, dst_hbm, z_hbm, out_hbm,
                    srcv, dstv, t2b, erb, sbuf, msgb, acc, sem1, sem2):
    cid = lax.axis_index("c")
    sid = lax.axis_index("s")
    wid = cid * 16 + sid
    _rowcopy(z_hbm, acc, sid)
    pltpu.sync_copy(src_hbm.at[wid], srcv)
    pltpu.sync_copy(dst_hbm.at[wid], dstv)
    plsc.subcore_barrier()

    il = lax.iota(jnp.int32, 16)
    col_el = jnp.full((16,), F2, jnp.int32)
    col0 = jnp.zeros((16,), jnp.int32)
    one = jnp.full((16,), 1.0, jnp.float32)
    zero = jnp.zeros((16,), jnp.float32)

    @pl.loop(0, NCH)
    def _chunk(j):
        g1 = pltpu.async_copy(t2_hbm.at[srcv.at[j]], t2b, sem1)
        g2 = pltpu.async_copy(er_hbm.at[dstv.at[j]], erb, sem2)
        g1.wait()
        g2.wait()

        @pl.loop(0, CHUNK // 16)
        def _sgrp(g):
            rows = g * 16 + il
            elv = plsc.load_gather(t2b, (rows, col_el))
            erv = plsc.load_gather(erb, (rows, col0))
            ev = elv + erv
            ev = jnp.where(ev >= 0.0, ev, 0.2 * ev)
            sbuf[pl.ds(g * 16, 16)] = jnp.exp(ev)

        @pl.loop(0, CHUNK)
        def _edge(e):
            sb = plsc.load_gather(sbuf, (jnp.full((16,), e, jnp.int32),))
            for q in range(3):
                tv = t2b[e, pl.ds(16 * q, 16)]
                mv = tv * sb
                if q == 2:
                    mv = jnp.where(il < 8, mv,
                                   jnp.where(il == 8, sb, zero))
                msgb[e, pl.ds(16 * q, 16)] = mv

        pltpu.sync_copy(msgb, acc.at[dstv.at[j]], add=True)

    plsc.subcore_barrier()
    _rowcopy(acc, out_hbm.at[cid], sid)


def _sc_layer2(t2, er2, src_r, dst_r, z48):
    k = pl.kernel(
        _sc_layer2_body,
        out_type=jax.ShapeDtypeStruct((2, N, TW2), jnp.float32),
        mesh=_mesh(),
        compiler_params=_sc_params(),
        scratch_types=[
            pltpu.VMEM((NCH, CHUNK), jnp.int32),
            pltpu.VMEM((NCH, CHUNK), jnp.int32),
            pltpu.VMEM((CHUNK, TW2), jnp.float32),
            pltpu.VMEM((CHUNK, ERW), jnp.float32),
            pltpu.VMEM((CHUNK,), jnp.float32),
            pltpu.VMEM((CHUNK, TW2), jnp.float32),
            pltpu.VMEM_SHARED((N, TW2), jnp.float32),
            pltpu.SemaphoreType.DMA,
            pltpu.SemaphoreType.DMA,
        ],
    )
    return k(t2, er2, src_r, dst_r, z48)


def kernel(x, edge_index, W1, al1, ar1, b1, W2, al2, ar2, b2):
    w1r = W1.reshape(H1, F1, F_IN)
    a_l1 = jnp.einsum("hfk,hf->kh", w1r, al1[0])
    a_r1 = jnp.einsum("hfk,hf->kh", w1r, ar1[0])
    zc8 = jnp.zeros((F_IN, 8), jnp.float32)
    wc1 = jnp.concatenate([W1.T, a_l1, zc8, a_r1, zc8], axis=1)

    w2r = W2.reshape(H2, F2, D1)
    a_l2 = jnp.einsum("hfk,hf->kh", w2r, al2[0])
    a_r2 = jnp.einsum("hfk,hf->kh", w2r, ar2[0])
    zc7 = jnp.zeros((D1, 7), jnp.float32)
    zc15 = jnp.zeros((D1, 15), jnp.float32)
    wc2 = jnp.concatenate([W2.T, a_l2, zc7, a_r2, zc15], axis=1)

    src_r = edge_index[0].reshape(NTILES, NCH, CHUNK)
    dst_r = edge_index[1].reshape(NTILES, NCH, CHUNK)
    z80 = jnp.zeros((N, TW1), jnp.float32)
    z48 = jnp.zeros((N, TW2), jnp.float32)
    b1row = b1.reshape(1, D1)
    b2row = b2.reshape(1, F2)

    t1, er1 = _tc_project(x, wc1, TW1)
    p1 = _sc_layer1(t1, er1, src_r, dst_r, z80)
    t2, er2 = _tc_mid(p1, wc2, b1row)
    p2 = _sc_layer2(t2, er2, src_r, dst_r, z48)
    return _tc_final(p2, b2row)


# speedup vs baseline: 87.8807x; 1.6159x over previous
import dataclasses
import functools

import jax
import jax.numpy as jnp
from jax import lax
from jax.experimental import pallas as pl
from jax.experimental.pallas import tpu as pltpu
from jax.experimental.pallas import tpu_sc as plsc

N = 10000
E = 320000
F_IN = 128
H1, F1 = 8, 8
D1 = H1 * F1
H2, F2 = 1, 40
D2 = H2 * F2

TW1 = 80
TW2 = 48
ERW = 16

NTILES = 32
CHUNK = 125
EPT = E // NTILES
NCH = EPT // CHUNK
NBUF = 2
R0 = 624
RLAST = N - 15 * R0

def _vgather(x, idx):
    dnums = lax.GatherDimensionNumbers(
        offset_dims=(), collapsed_slice_dims=(0,), start_index_map=(0,))
    return lax.gather(x, idx[:, None], dnums, (1,),
                      mode=lax.GatherScatterMode.PROMISE_IN_BOUNDS)


def _mesh():
    return plsc.VectorSubcoreMesh(core_axis_name="c", subcore_axis_name="s")


def _sc_params():
    cp = pltpu.CompilerParams()
    fields = pltpu.CompilerParams.__dataclass_fields__
    if "needs_layout_passes" in fields:
        cp = dataclasses.replace(cp, needs_layout_passes=False)
    if "use_tc_tiling_on_sc" in fields:
        cp = dataclasses.replace(cp, use_tc_tiling_on_sc=False)
    return cp


def _mm_body(x_ref, w_ref, a_ref, b_ref, split):
    y = jnp.dot(x_ref[...], w_ref[...], preferred_element_type=jnp.float32)
    a_ref[...] = y[:, :split]
    b_ref[...] = y[:, split:]


def _tc_project(x, w, split, rows_per_blk=1000):
    n, k = x.shape
    m = w.shape[1]
    grid = (n // rows_per_blk,)
    return pl.pallas_call(
        functools.partial(_mm_body, split=split),
        grid=grid,
        in_specs=[
            pl.BlockSpec((rows_per_blk, k), lambda i: (i, 0)),
            pl.BlockSpec((k, m), lambda i: (0, 0)),
        ],
        out_specs=[
            pl.BlockSpec((rows_per_blk, split), lambda i: (i, 0)),
            pl.BlockSpec((rows_per_blk, m - split), lambda i: (i, 0)),
        ],
        out_shape=[
            jax.ShapeDtypeStruct((n, split), jnp.float32),
            jax.ShapeDtypeStruct((n, m - split), jnp.float32),
        ],
    )(x, w)


def _mid_body(p_ref, w_ref, b_ref, t2_ref, er_ref):
    a = p_ref[0] + p_ref[1]
    pieces = []
    for h in range(H1):
        d = a[:, D1 + h:D1 + h + 1]
        d = jnp.where(d != 0.0, d, 1.0)
        pieces.append(a[:, F1 * h:F1 * h + F1] / d)
    o = jnp.concatenate(pieces, axis=1)
    hb = jnp.maximum(o + b_ref[...], 0.0)
    y = jnp.dot(hb, w_ref[...], preferred_element_type=jnp.float32)
    t2_ref[...] = y[:, :TW2]
    er_ref[...] = y[:, TW2:]


def _tc_mid(p1, w, b1row, rows_per_blk=1000):
    grid = (N // rows_per_blk,)
    return pl.pallas_call(
        _mid_body,
        grid=grid,
        in_specs=[
            pl.BlockSpec((2, rows_per_blk, TW1), lambda i: (0, i, 0)),
            pl.BlockSpec(w.shape, lambda i: (0, 0)),
            pl.BlockSpec((1, D1), lambda i: (0, 0)),
        ],
        out_specs=[
            pl.BlockSpec((rows_per_blk, TW2), lambda i: (i, 0)),
            pl.BlockSpec((rows_per_blk, ERW), lambda i: (i, 0)),
        ],
        out_shape=[
            jax.ShapeDtypeStruct((N, TW2), jnp.float32),
            jax.ShapeDtypeStruct((N, ERW), jnp.float32),
        ],
    )(p1, w, b1row)


def _final_body(p_ref, b_ref, o_ref):
    a = p_ref[0] + p_ref[1]
    d = a[:, F2:F2 + 1]
    d = jnp.where(d != 0.0, d, 1.0)
    z = a[:, :F2] / d + b_ref[...]
    m = jnp.max(z, axis=1, keepdims=True)
    lse = m + jnp.log(jnp.sum(jnp.exp(z - m), axis=1, keepdims=True))
    o_ref[...] = z - lse


def _tc_final(p2, b2row, rows_per_blk=1000):
    grid = (N // rows_per_blk,)
    return pl.pallas_call(
        _final_body,
        grid=grid,
        in_specs=[
            pl.BlockSpec((2, rows_per_blk, TW2), lambda i: (0, i, 0)),
            pl.BlockSpec((1, F2), lambda i: (0, 0)),
        ],
        out_specs=pl.BlockSpec((rows_per_blk, F2), lambda i: (i, 0)),
        out_shape=jax.ShapeDtypeStruct((N, F2), jnp.float32),
    )(p2, b2row)


def _sc_layer1_body(t1_hbm, er_hbm, src_hbm, dst_hbm, z_hbm, out_hbm,
                    srcv, dstv, t1b, erb, msgb, acc,
                    gsem1, gsem2, ssem):
    cid = lax.axis_index("c")
    sid = lax.axis_index("s")
    wid = cid * 16 + sid
    _rowcopy(z_hbm, acc, sid)
    pltpu.sync_copy(src_hbm.at[wid], srcv)
    pltpu.sync_copy(dst_hbm.at[wid], dstv)
    plsc.subcore_barrier()

    il = lax.iota(jnp.int32, 16)
    head_sel = il >> 3
    takeidx = [2 * q + head_sel for q in range(4)]

    def gathers(c, b):
        pltpu.async_copy(t1_hbm.at[srcv.at[c]], t1b.at[b], gsem1.at[b])
        pltpu.async_copy(er_hbm.at[dstv.at[c]], erb.at[b], gsem2.at[b])

    for b in range(NBUF):
        gathers(b, b)

    @pl.loop(0, NCH, step=NBUF)
    def _chunk(j):
        for b in range(NBUF):
            c = j + b
            pltpu.make_async_copy(t1_hbm.at[srcv.at[c]],
                                  t1b.at[b], gsem1.at[b]).wait()
            pltpu.make_async_copy(er_hbm.at[dstv.at[c]],
                                  erb.at[b], gsem2.at[b]).wait()

            @pl.when(c >= NBUF)
            def _():
                pltpu.make_async_copy(msgb.at[b], acc.at[dstv.at[c]],
                                      ssem.at[b]).wait()

            @pl.loop(0, CHUNK)
            def _edge(e):
                elv = t1b[b, e, pl.ds(D1, 16)]
                erv = erb[b, e, :]
                ev = elv + erv
                ev = jnp.where(ev >= 0.0, ev, 0.2 * ev)
                sv = jnp.exp(ev)
                msgb[b, e, pl.ds(D1, 16)] = sv
                for q in range(4):
                    hv = t1b[b, e, pl.ds(16 * q, 16)]
                    sb = _vgather(sv, takeidx[q])
                    msgb[b, e, pl.ds(16 * q, 16)] = hv * sb

            pltpu.async_copy(msgb.at[b], acc.at[dstv.at[c]], ssem.at[b],
                             add=True)

            @pl.when(c + NBUF < NCH)
            def _():
                gathers(c + NBUF, b)

    for b in range(NBUF):
        pltpu.make_async_copy(msgb.at[b], acc.at[pl.ds(0, CHUNK)],
                              ssem.at[b]).wait()

    plsc.subcore_barrier()
    _rowcopy(acc, out_hbm.at[cid], sid)


def _rowcopy(src, dst, sid):
    @pl.when(sid < 15)
    def _():
        st = pl.multiple_of(sid * R0, 8)
        pltpu.sync_copy(src.at[pl.ds(st, R0)], dst.at[pl.ds(st, R0)])

    @pl.when(sid == 15)
    def _():
        pltpu.sync_copy(src.at[pl.ds(15 * R0, RLAST)],
                        dst.at[pl.ds(15 * R0, RLAST)])


def _sc_layer1(t1, er1, src_r, dst_r, z80):
    k = pl.kernel(
        _sc_layer1_body,
        out_type=jax.ShapeDtypeStruct((2, N, TW1), jnp.float32),
        mesh=_mesh(),
        compiler_params=_sc_params(),
        scratch_types=[
            pltpu.VMEM((NCH, CHUNK), jnp.int32),
            pltpu.VMEM((NCH, CHUNK), jnp.int32),
            pltpu.VMEM((NBUF, CHUNK, TW1), jnp.float32),
            pltpu.VMEM((NBUF, CHUNK, ERW), jnp.float32),
            pltpu.VMEM((NBUF, CHUNK, TW1), jnp.float32),
            pltpu.VMEM_SHARED((N, TW1), jnp.float32),
            pltpu.SemaphoreType.DMA((NBUF,)),
            pltpu.SemaphoreType.DMA((NBUF,)),
            pltpu.SemaphoreType.DMA((NBUF,)),
        ],
    )
    return k(t1, er1, src_r, dst_r, z80)


def _sc_layer2_body(t2_hbm, er_hbm, src_hbm, dst_hbm, z_hbm, out_hbm,
                    srcv, dstv, t2b, erb, msgb, acc,
                    gsem1, gsem2, ssem):
    cid = lax.axis_index("c")
    sid = lax.axis_index("s")
    wid = cid * 16 + sid
    _rowcopy(z_hbm, acc, sid)
    pltpu.sync_copy(src_hbm.at[wid], srcv)
    pltpu.sync_copy(dst_hbm.at[wid], dstv)
    plsc.subcore_barrier()

    il = lax.iota(jnp.int32, 16)
    zero = jnp.zeros((16,), jnp.float32)

    def gathers(c, b):
        pltpu.async_copy(t2_hbm.at[srcv.at[c]], t2b.at[b], gsem1.at[b])
        pltpu.async_copy(er_hbm.at[dstv.at[c]], erb.at[b], gsem2.at[b])

    for b in range(NBUF):
        gathers(b, b)

    @pl.loop(0, NCH, step=NBUF)
    def _chunk(j):
        for b in range(NBUF):
            c = j + b
            pltpu.make_async_copy(t2_hbm.at[srcv.at[c]],
                                  t2b.at[b], gsem1.at[b]).wait()
            pltpu.make_async_copy(er_hbm.at[dstv.at[c]],
                                  erb.at[b], gsem2.at[b]).wait()

            @pl.when(c >= NBUF)
            def _():
                pltpu.make_async_copy(msgb.at[b], acc.at[dstv.at[c]],
                                      ssem.at[b]).wait()

            @pl.loop(0, CHUNK)
            def _edge(e):
                eb = plsc.load_gather(
                    t2b.at[b], (jnp.full((16,), e, jnp.int32),
                                jnp.full((16,), F2, jnp.int32)))
                rb = plsc.load_gather(
                    erb.at[b], (jnp.full((16,), e, jnp.int32),
                                jnp.zeros((16,), jnp.int32)))
                ev = eb + rb
                ev = jnp.where(ev >= 0.0, ev, 0.2 * ev)
                sb = jnp.exp(ev)
                for q in range(3):
                    tv = t2b[b, e, pl.ds(16 * q, 16)]
                    mv = tv * sb
                    if q == 2:
                        mv = jnp.where(il < 8, mv,
                                       jnp.where(il == 8, sb, zero))
                    msgb[b, e, pl.ds(16 * q, 16)] = mv

            pltpu.async_copy(msgb.at[b], acc.at[dstv.at[c]], ssem.at[b],
                             add=True)

            @pl.when(c + NBUF < NCH)
            def _():
                gathers(c + NBUF, b)

    for b in range(NBUF):
        pltpu.make_async_copy(msgb.at[b], acc.at[pl.ds(0, CHUNK)],
                              ssem.at[b]).wait()

    plsc.subcore_barrier()
    _rowcopy(acc, out_hbm.at[cid], sid)


def _sc_layer2(t2, er2, src_r, dst_r, z48):
    k = pl.kernel(
        _sc_layer2_body,
        out_type=jax.ShapeDtypeStruct((2, N, TW2), jnp.float32),
        mesh=_mesh(),
        compiler_params=_sc_params(),
        scratch_types=[
            pltpu.VMEM((NCH, CHUNK), jnp.int32),
            pltpu.VMEM((NCH, CHUNK), jnp.int32),
            pltpu.VMEM((NBUF, CHUNK, TW2), jnp.float32),
            pltpu.VMEM((NBUF, CHUNK, ERW), jnp.float32),
            pltpu.VMEM((NBUF, CHUNK, TW2), jnp.float32),
            pltpu.VMEM_SHARED((N, TW2), jnp.float32),
            pltpu.SemaphoreType.DMA((NBUF,)),
            pltpu.SemaphoreType.DMA((NBUF,)),
            pltpu.SemaphoreType.DMA((NBUF,)),
        ],
    )
    return k(t2, er2, src_r, dst_r, z48)


def kernel(x, edge_index, W1, al1, ar1, b1, W2, al2, ar2, b2):
    w1r = W1.reshape(H1, F1, F_IN)
    a_l1 = jnp.einsum("hfk,hf->kh", w1r, al1[0])
    a_r1 = jnp.einsum("hfk,hf->kh", w1r, ar1[0])
    zc8 = jnp.zeros((F_IN, 8), jnp.float32)
    wc1 = jnp.concatenate([W1.T, a_l1, zc8, a_r1, zc8], axis=1)

    w2r = W2.reshape(H2, F2, D1)
    a_l2 = jnp.einsum("hfk,hf->kh", w2r, al2[0])
    a_r2 = jnp.einsum("hfk,hf->kh", w2r, ar2[0])
    zc7 = jnp.zeros((D1, 7), jnp.float32)
    zc15 = jnp.zeros((D1, 15), jnp.float32)
    wc2 = jnp.concatenate([W2.T, a_l2, zc7, a_r2, zc15], axis=1)

    src_r = edge_index[0].reshape(NTILES, NCH, CHUNK)
    dst_r = edge_index[1].reshape(NTILES, NCH, CHUNK)
    z80 = jnp.zeros((N, TW1), jnp.float32)
    z48 = jnp.zeros((N, TW2), jnp.float32)
    b1row = b1.reshape(1, D1)
    b2row = b2.reshape(1, F2)

    t1, er1 = _tc_project(x, wc1, TW1)
    p1 = _sc_layer1(t1, er1, src_r, dst_r, z80)
    t2, er2 = _tc_mid(p1, wc2, b1row)
    p2 = _sc_layer2(t2, er2, src_r, dst_r, z48)
    return _tc_final(p2, b2row)


# speedup vs baseline: 178.6966x; 2.0334x over previous
import dataclasses
import functools

import jax
import jax.numpy as jnp
from jax import lax
from jax.experimental import pallas as pl
from jax.experimental.pallas import tpu as pltpu
from jax.experimental.pallas import tpu_sc as plsc

N = 10000
E = 320000
F_IN = 128
H1, F1 = 8, 8
D1 = H1 * F1
H2, F2 = 1, 40
D2 = H2 * F2

TW1 = 80
TW2 = 48
ERW = 16

NTILES = 32
CHUNK = 125
EPT = E // NTILES
NCH = EPT // CHUNK
NBUF = 2
R0 = 624
RLAST = N - 15 * R0

def _vgather(x, idx):
    dnums = lax.GatherDimensionNumbers(
        offset_dims=(), collapsed_slice_dims=(0,), start_index_map=(0,))
    return lax.gather(x, idx[:, None], dnums, (1,),
                      mode=lax.GatherScatterMode.PROMISE_IN_BOUNDS)


def _mesh():
    return plsc.VectorSubcoreMesh(core_axis_name="c", subcore_axis_name="s")


def _sc_params():
    cp = pltpu.CompilerParams()
    fields = pltpu.CompilerParams.__dataclass_fields__
    if "needs_layout_passes" in fields:
        cp = dataclasses.replace(cp, needs_layout_passes=False)
    if "use_tc_tiling_on_sc" in fields:
        cp = dataclasses.replace(cp, use_tc_tiling_on_sc=False)
    return cp


def _mm_body(x_ref, w_ref, a_ref, b_ref, split):
    y = jnp.dot(x_ref[...], w_ref[...], preferred_element_type=jnp.float32)
    a_ref[...] = y[:, :split]
    b_ref[...] = y[:, split:]


def _tc_project(x, w, split, rows_per_blk=1000):
    n, k = x.shape
    m = w.shape[1]
    grid = (n // rows_per_blk,)
    return pl.pallas_call(
        functools.partial(_mm_body, split=split),
        grid=grid,
        in_specs=[
            pl.BlockSpec((rows_per_blk, k), lambda i: (i, 0)),
            pl.BlockSpec((k, m), lambda i: (0, 0)),
        ],
        out_specs=[
            pl.BlockSpec((rows_per_blk, split), lambda i: (i, 0)),
            pl.BlockSpec((rows_per_blk, m - split), lambda i: (i, 0)),
        ],
        out_shape=[
            jax.ShapeDtypeStruct((n, split), jnp.float32),
            jax.ShapeDtypeStruct((n, m - split), jnp.float32),
        ],
    )(x, w)


def _mid_body(p_ref, w_ref, b_ref, t2_ref, er_ref):
    a = p_ref[0] + p_ref[1]
    pieces = []
    for h in range(H1):
        d = a[:, D1 + h:D1 + h + 1]
        d = jnp.where(d != 0.0, d, 1.0)
        pieces.append(a[:, F1 * h:F1 * h + F1] / d)
    o = jnp.concatenate(pieces, axis=1)
    hb = jnp.maximum(o + b_ref[...], 0.0)
    y = jnp.dot(hb, w_ref[...], preferred_element_type=jnp.float32)
    t2_ref[...] = y[:, :TW2]
    er_ref[...] = y[:, TW2:]


def _tc_mid(p1, w, b1row, rows_per_blk=1000):
    grid = (N // rows_per_blk,)
    return pl.pallas_call(
        _mid_body,
        grid=grid,
        in_specs=[
            pl.BlockSpec((2, rows_per_blk, TW1), lambda i: (0, i, 0)),
            pl.BlockSpec(w.shape, lambda i: (0, 0)),
            pl.BlockSpec((1, D1), lambda i: (0, 0)),
        ],
        out_specs=[
            pl.BlockSpec((rows_per_blk, TW2), lambda i: (i, 0)),
            pl.BlockSpec((rows_per_blk, ERW), lambda i: (i, 0)),
        ],
        out_shape=[
            jax.ShapeDtypeStruct((N, TW2), jnp.float32),
            jax.ShapeDtypeStruct((N, ERW), jnp.float32),
        ],
    )(p1, w, b1row)


def _final_body(p_ref, b_ref, o_ref):
    a = p_ref[0] + p_ref[1]
    d = a[:, F2:F2 + 1]
    d = jnp.where(d != 0.0, d, 1.0)
    z = a[:, :F2] / d + b_ref[...]
    m = jnp.max(z, axis=1, keepdims=True)
    lse = m + jnp.log(jnp.sum(jnp.exp(z - m), axis=1, keepdims=True))
    o_ref[...] = z - lse


def _tc_final(p2, b2row, rows_per_blk=1000):
    grid = (N // rows_per_blk,)
    return pl.pallas_call(
        _final_body,
        grid=grid,
        in_specs=[
            pl.BlockSpec((2, rows_per_blk, TW2), lambda i: (0, i, 0)),
            pl.BlockSpec((1, F2), lambda i: (0, 0)),
        ],
        out_specs=pl.BlockSpec((rows_per_blk, F2), lambda i: (i, 0)),
        out_shape=jax.ShapeDtypeStruct((N, F2), jnp.float32),
    )(p2, b2row)


def _sc_layer1_body(t1_hbm, er_hbm, src_hbm, dst_hbm, z_hbm, out_hbm,
                    srcv, dstv, t1b, erb, msgb, acc,
                    gsem1, gsem2, ssem):
    cid = lax.axis_index("c")
    sid = lax.axis_index("s")
    wid = cid * 16 + sid
    _rowcopy(z_hbm, acc, sid)
    pltpu.sync_copy(src_hbm.at[wid], srcv)
    pltpu.sync_copy(dst_hbm.at[wid], dstv)
    plsc.subcore_barrier()

    il = lax.iota(jnp.int32, 16)
    head_sel = il >> 3
    takeidx = [2 * q + head_sel for q in range(4)]

    def gathers(c, b):
        pltpu.async_copy(t1_hbm.at[srcv.at[c]], t1b.at[b], gsem1.at[b])
        pltpu.async_copy(er_hbm.at[dstv.at[c]], erb.at[b], gsem2.at[b])

    for b in range(NBUF):
        gathers(b, b)

    @pl.loop(0, NCH, step=NBUF)
    def _chunk(j):
        for b in range(NBUF):
            c = j + b
            pltpu.make_async_copy(t1_hbm.at[srcv.at[c]],
                                  t1b.at[b], gsem1.at[b]).wait()
            pltpu.make_async_copy(er_hbm.at[dstv.at[c]],
                                  erb.at[b], gsem2.at[b]).wait()

            @pl.when(c >= NBUF)
            def _():
                pltpu.make_async_copy(msgb.at[b], acc.at[dstv.at[c]],
                                      ssem.at[b]).wait()

            @plsc.parallel_loop(0, CHUNK, unroll=4)
            def _edge(e):
                elv = t1b[b, e, pl.ds(D1, 16)]
                erv = erb[b, e, :]
                ev = elv + erv
                ev = jnp.where(ev >= 0.0, ev, 0.2 * ev)
                sv = jnp.exp(ev)
                msgb[b, e, pl.ds(D1, 16)] = sv
                for q in range(4):
                    hv = t1b[b, e, pl.ds(16 * q, 16)]
                    sb = _vgather(sv, takeidx[q])
                    msgb[b, e, pl.ds(16 * q, 16)] = hv * sb

            pltpu.async_copy(msgb.at[b], acc.at[dstv.at[c]], ssem.at[b],
                             add=True)

            @pl.when(c + NBUF < NCH)
            def _():
                gathers(c + NBUF, b)

    for b in range(NBUF):
        pltpu.make_async_copy(msgb.at[b], acc.at[pl.ds(0, CHUNK)],
                              ssem.at[b]).wait()

    plsc.subcore_barrier()
    _rowcopy(acc, out_hbm.at[cid], sid)


def _rowcopy(src, dst, sid):
    @pl.when(sid < 15)
    def _():
        st = pl.multiple_of(sid * R0, 8)
        pltpu.sync_copy(src.at[pl.ds(st, R0)], dst.at[pl.ds(st, R0)])

    @pl.when(sid == 15)
    def _():
        pltpu.sync_copy(src.at[pl.ds(15 * R0, RLAST)],
                        dst.at[pl.ds(15 * R0, RLAST)])


def _sc_layer1(t1, er1, src_r, dst_r, z80):
    k = pl.kernel(
        _sc_layer1_body,
        out_type=jax.ShapeDtypeStruct((2, N, TW1), jnp.float32),
        mesh=_mesh(),
        compiler_params=_sc_params(),
        scratch_types=[
            pltpu.VMEM((NCH, CHUNK), jnp.int32),
            pltpu.VMEM((NCH, CHUNK), jnp.int32),
            pltpu.VMEM((NBUF, CHUNK, TW1), jnp.float32),
            pltpu.VMEM((NBUF, CHUNK, ERW), jnp.float32),
            pltpu.VMEM((NBUF, CHUNK, TW1), jnp.float32),
            pltpu.VMEM_SHARED((N, TW1), jnp.float32),
            pltpu.SemaphoreType.DMA((NBUF,)),
            pltpu.SemaphoreType.DMA((NBUF,)),
            pltpu.SemaphoreType.DMA((NBUF,)),
        ],
    )
    return k(t1, er1, src_r, dst_r, z80)


def _sc_layer2_body(t2_hbm, er_hbm, src_hbm, dst_hbm, z_hbm, out_hbm,
                    srcv, dstv, t2b, erb, msgb, acc,
                    gsem1, gsem2, ssem):
    cid = lax.axis_index("c")
    sid = lax.axis_index("s")
    wid = cid * 16 + sid
    _rowcopy(z_hbm, acc, sid)
    pltpu.sync_copy(src_hbm.at[wid], srcv)
    pltpu.sync_copy(dst_hbm.at[wid], dstv)
    plsc.subcore_barrier()

    il = lax.iota(jnp.int32, 16)
    zero = jnp.zeros((16,), jnp.float32)

    def gathers(c, b):
        pltpu.async_copy(t2_hbm.at[srcv.at[c]], t2b.at[b], gsem1.at[b])
        pltpu.async_copy(er_hbm.at[dstv.at[c]], erb.at[b], gsem2.at[b])

    for b in range(NBUF):
        gathers(b, b)

    @pl.loop(0, NCH, step=NBUF)
    def _chunk(j):
        for b in range(NBUF):
            c = j + b
            pltpu.make_async_copy(t2_hbm.at[srcv.at[c]],
                                  t2b.at[b], gsem1.at[b]).wait()
            pltpu.make_async_copy(er_hbm.at[dstv.at[c]],
                                  erb.at[b], gsem2.at[b]).wait()

            @pl.when(c >= NBUF)
            def _():
                pltpu.make_async_copy(msgb.at[b], acc.at[dstv.at[c]],
                                      ssem.at[b]).wait()

            @plsc.parallel_loop(0, CHUNK, unroll=4)
            def _edge(e):
                eb = plsc.load_gather(
                    t2b.at[b], (jnp.full((16,), e, jnp.int32),
                                jnp.full((16,), F2, jnp.int32)))
                rb = plsc.load_gather(
                    erb.at[b], (jnp.full((16,), e, jnp.int32),
                                jnp.zeros((16,), jnp.int32)))
                ev = eb + rb
                ev = jnp.where(ev >= 0.0, ev, 0.2 * ev)
                sb = jnp.exp(ev)
                for q in range(3):
                    tv = t2b[b, e, pl.ds(16 * q, 16)]
                    mv = tv * sb
                    if q == 2:
                        mv = jnp.where(il < 8, mv,
                                       jnp.where(il == 8, sb, zero))
                    msgb[b, e, pl.ds(16 * q, 16)] = mv

            pltpu.async_copy(msgb.at[b], acc.at[dstv.at[c]], ssem.at[b],
                             add=True)

            @pl.when(c + NBUF < NCH)
            def _():
                gathers(c + NBUF, b)

    for b in range(NBUF):
        pltpu.make_async_copy(msgb.at[b], acc.at[pl.ds(0, CHUNK)],
                              ssem.at[b]).wait()

    plsc.subcore_barrier()
    _rowcopy(acc, out_hbm.at[cid], sid)


def _sc_layer2(t2, er2, src_r, dst_r, z48):
    k = pl.kernel(
        _sc_layer2_body,
        out_type=jax.ShapeDtypeStruct((2, N, TW2), jnp.float32),
        mesh=_mesh(),
        compiler_params=_sc_params(),
        scratch_types=[
            pltpu.VMEM((NCH, CHUNK), jnp.int32),
            pltpu.VMEM((NCH, CHUNK), jnp.int32),
            pltpu.VMEM((NBUF, CHUNK, TW2), jnp.float32),
            pltpu.VMEM((NBUF, CHUNK, ERW), jnp.float32),
            pltpu.VMEM((NBUF, CHUNK, TW2), jnp.float32),
            pltpu.VMEM_SHARED((N, TW2), jnp.float32),
            pltpu.SemaphoreType.DMA((NBUF,)),
            pltpu.SemaphoreType.DMA((NBUF,)),
            pltpu.SemaphoreType.DMA((NBUF,)),
        ],
    )
    return k(t2, er2, src_r, dst_r, z48)


def kernel(x, edge_index, W1, al1, ar1, b1, W2, al2, ar2, b2):
    w1r = W1.reshape(H1, F1, F_IN)
    a_l1 = jnp.einsum("hfk,hf->kh", w1r, al1[0])
    a_r1 = jnp.einsum("hfk,hf->kh", w1r, ar1[0])
    zc8 = jnp.zeros((F_IN, 8), jnp.float32)
    wc1 = jnp.concatenate([W1.T, a_l1, zc8, a_r1, zc8], axis=1)

    w2r = W2.reshape(H2, F2, D1)
    a_l2 = jnp.einsum("hfk,hf->kh", w2r, al2[0])
    a_r2 = jnp.einsum("hfk,hf->kh", w2r, ar2[0])
    zc7 = jnp.zeros((D1, 7), jnp.float32)
    zc15 = jnp.zeros((D1, 15), jnp.float32)
    wc2 = jnp.concatenate([W2.T, a_l2, zc7, a_r2, zc15], axis=1)

    src_r = edge_index[0].reshape(NTILES, NCH, CHUNK)
    dst_r = edge_index[1].reshape(NTILES, NCH, CHUNK)
    z80 = jnp.zeros((N, TW1), jnp.float32)
    z48 = jnp.zeros((N, TW2), jnp.float32)
    b1row = b1.reshape(1, D1)
    b2row = b2.reshape(1, F2)

    t1, er1 = _tc_project(x, wc1, TW1)
    p1 = _sc_layer1(t1, er1, src_r, dst_r, z80)
    t2, er2 = _tc_mid(p1, wc2, b1row)
    p2 = _sc_layer2(t2, er2, src_r, dst_r, z48)
    return _tc_final(p2, b2row)


# speedup vs baseline: 182.7807x; 1.0229x over previous
import dataclasses
import functools

import jax
import jax.numpy as jnp
from jax import lax
from jax.experimental import pallas as pl
from jax.experimental.pallas import tpu as pltpu
from jax.experimental.pallas import tpu_sc as plsc

N = 10000
E = 320000
F_IN = 128
H1, F1 = 8, 8
D1 = H1 * F1
H2, F2 = 1, 40
D2 = H2 * F2

TW1 = 80
TW2 = 48
GW1 = 96
GW2 = 64
ERW = 16

NTILES = 32
CHUNK = 125
EPT = E // NTILES
NCH = EPT // CHUNK
NBUF = 2
R0 = 624
RLAST = N - 15 * R0

def _bf16_perm(width):
    p = []
    for blk in range(width // 32):
        for i in range(16):
            p.append(32 * blk + i)
            p.append(32 * blk + 16 + i)
    return p


def _unpack32(v):
    return plsc.unpack(v, format=plsc.PackFormat.INTERLEAVED,
                       preferred_element_type=jnp.float32)


def _vgather(x, idx):
    dnums = lax.GatherDimensionNumbers(
        offset_dims=(), collapsed_slice_dims=(0,), start_index_map=(0,))
    return lax.gather(x, idx[:, None], dnums, (1,),
                      mode=lax.GatherScatterMode.PROMISE_IN_BOUNDS)


def _mesh():
    return plsc.VectorSubcoreMesh(core_axis_name="c", subcore_axis_name="s")


def _sc_params():
    cp = pltpu.CompilerParams()
    fields = pltpu.CompilerParams.__dataclass_fields__
    if "needs_layout_passes" in fields:
        cp = dataclasses.replace(cp, needs_layout_passes=False)
    if "use_tc_tiling_on_sc" in fields:
        cp = dataclasses.replace(cp, use_tc_tiling_on_sc=False)
    return cp


def _mm_body(x_ref, w_ref, a_ref, b_ref, split):
    y = jnp.dot(x_ref[...], w_ref[...], preferred_element_type=jnp.float32)
    a_ref[...] = y[:, :split].astype(a_ref.dtype)
    b_ref[...] = y[:, split:]


def _tc_project(x, w, split, rows_per_blk=1000):
    n, k = x.shape
    m = w.shape[1]
    grid = (n // rows_per_blk,)
    return pl.pallas_call(
        functools.partial(_mm_body, split=split),
        grid=grid,
        in_specs=[
            pl.BlockSpec((rows_per_blk, k), lambda i: (i, 0)),
            pl.BlockSpec((k, m), lambda i: (0, 0)),
        ],
        out_specs=[
            pl.BlockSpec((rows_per_blk, split), lambda i: (i, 0)),
            pl.BlockSpec((rows_per_blk, m - split), lambda i: (i, 0)),
        ],
        out_shape=[
            jax.ShapeDtypeStruct((n, split), jnp.bfloat16),
            jax.ShapeDtypeStruct((n, m - split), jnp.float32),
        ],
    )(x, w)


def _mid_body(p_ref, w_ref, b_ref, t2_ref, er_ref):
    a = p_ref[0] + p_ref[1]
    pieces = []
    for h in range(H1):
        d = a[:, D1 + h:D1 + h + 1]
        d = jnp.where(d != 0.0, d, 1.0)
        pieces.append(a[:, F1 * h:F1 * h + F1] / d)
    o = jnp.concatenate(pieces, axis=1)
    hb = jnp.maximum(o + b_ref[...], 0.0)
    y = jnp.dot(hb, w_ref[...], preferred_element_type=jnp.float32)
    t2_ref[...] = y[:, :GW2].astype(jnp.bfloat16)
    er_ref[...] = y[:, GW2:]


def _tc_mid(p1, w, b1row, rows_per_blk=1000):
    grid = (N // rows_per_blk,)
    return pl.pallas_call(
        _mid_body,
        grid=grid,
        in_specs=[
            pl.BlockSpec((2, rows_per_blk, TW1), lambda i: (0, i, 0)),
            pl.BlockSpec(w.shape, lambda i: (0, 0)),
            pl.BlockSpec((1, D1), lambda i: (0, 0)),
        ],
        out_specs=[
            pl.BlockSpec((rows_per_blk, GW2), lambda i: (i, 0)),
            pl.BlockSpec((rows_per_blk, ERW), lambda i: (i, 0)),
        ],
        out_shape=[
            jax.ShapeDtypeStruct((N, GW2), jnp.bfloat16),
            jax.ShapeDtypeStruct((N, ERW), jnp.float32),
        ],
    )(p1, w, b1row)


def _final_body(p_ref, b_ref, o_ref):
    a = p_ref[0] + p_ref[1]
    d = a[:, F2:F2 + 1]
    d = jnp.where(d != 0.0, d, 1.0)
    z = a[:, :F2] / d + b_ref[...]
    m = jnp.max(z, axis=1, keepdims=True)
    lse = m + jnp.log(jnp.sum(jnp.exp(z - m), axis=1, keepdims=True))
    o_ref[...] = z - lse


def _tc_final(p2, b2row, rows_per_blk=1000):
    grid = (N // rows_per_blk,)
    return pl.pallas_call(
        _final_body,
        grid=grid,
        in_specs=[
            pl.BlockSpec((2, rows_per_blk, TW2), lambda i: (0, i, 0)),
            pl.BlockSpec((1, F2), lambda i: (0, 0)),
        ],
        out_specs=pl.BlockSpec((rows_per_blk, F2), lambda i: (i, 0)),
        out_shape=jax.ShapeDtypeStruct((N, F2), jnp.float32),
    )(p2, b2row)


def _sc_layer1_body(t1_hbm, er_hbm, src_hbm, dst_hbm, z_hbm, out_hbm,
                    srcv, dstv, t1b, erb, msgb, acc,
                    gsem1, gsem2, ssem):
    cid = lax.axis_index("c")
    sid = lax.axis_index("s")
    wid = cid * 16 + sid
    _rowcopy(z_hbm, acc, sid)
    pltpu.sync_copy(src_hbm.at[wid], srcv)
    pltpu.sync_copy(dst_hbm.at[wid], dstv)
    plsc.subcore_barrier()

    il = lax.iota(jnp.int32, 16)
    head_sel = il >> 3
    takeidx = [2 * q + head_sel for q in range(4)]

    def gathers(c, b):
        pltpu.async_copy(t1_hbm.at[srcv.at[c]], t1b.at[b], gsem1.at[b])
        pltpu.async_copy(er_hbm.at[dstv.at[c]], erb.at[b], gsem2.at[b])

    for b in range(NBUF):
        gathers(b, b)

    @pl.loop(0, NCH, step=NBUF)
    def _chunk(j):
        for b in range(NBUF):
            c = j + b
            pltpu.make_async_copy(t1_hbm.at[srcv.at[c]],
                                  t1b.at[b], gsem1.at[b]).wait()
            pltpu.make_async_copy(er_hbm.at[dstv.at[c]],
                                  erb.at[b], gsem2.at[b]).wait()

            @pl.when(c >= NBUF)
            def _():
                pltpu.make_async_copy(msgb.at[b], acc.at[dstv.at[c]],
                                      ssem.at[b]).wait()

            @plsc.parallel_loop(0, CHUNK, unroll=4)
            def _edge(e):
                elv, _ = _unpack32(t1b[b, e, pl.ds(D1, 32)])
                erv = erb[b, e, :]
                ev = elv + erv
                ev = jnp.where(ev >= 0.0, ev, 0.2 * ev)
                sv = jnp.exp(ev)
                msgb[b, e, pl.ds(D1, 16)] = sv
                for q2 in range(2):
                    ha, hbv = _unpack32(t1b[b, e, pl.ds(32 * q2, 32)])
                    msgb[b, e, pl.ds(32 * q2, 16)] = (
                        ha * _vgather(sv, takeidx[2 * q2]))
                    msgb[b, e, pl.ds(32 * q2 + 16, 16)] = (
                        hbv * _vgather(sv, takeidx[2 * q2 + 1]))

            pltpu.async_copy(msgb.at[b], acc.at[dstv.at[c]], ssem.at[b],
                             add=True)

            @pl.when(c + NBUF < NCH)
            def _():
                gathers(c + NBUF, b)

    for b in range(NBUF):
        pltpu.make_async_copy(msgb.at[b], acc.at[pl.ds(0, CHUNK)],
                              ssem.at[b]).wait()

    plsc.subcore_barrier()
    _rowcopy(acc, out_hbm.at[cid], sid)


def _rowcopy(src, dst, sid):
    @pl.when(sid < 15)
    def _():
        st = pl.multiple_of(sid * R0, 8)
        pltpu.sync_copy(src.at[pl.ds(st, R0)], dst.at[pl.ds(st, R0)])

    @pl.when(sid == 15)
    def _():
        pltpu.sync_copy(src.at[pl.ds(15 * R0, RLAST)],
                        dst.at[pl.ds(15 * R0, RLAST)])


def _sc_layer1(t1, er1, src_r, dst_r, z80):
    k = pl.kernel(
        _sc_layer1_body,
        out_type=jax.ShapeDtypeStruct((2, N, TW1), jnp.float32),
        mesh=_mesh(),
        compiler_params=_sc_params(),
        scratch_types=[
            pltpu.VMEM((NCH, CHUNK), jnp.int32),
            pltpu.VMEM((NCH, CHUNK), jnp.int32),
            pltpu.VMEM((NBUF, CHUNK, GW1), jnp.bfloat16),
            pltpu.VMEM((NBUF, CHUNK, ERW), jnp.float32),
            pltpu.VMEM((NBUF, CHUNK, TW1), jnp.float32),
            pltpu.VMEM_SHARED((N, TW1), jnp.float32),
            pltpu.SemaphoreType.DMA((NBUF,)),
            pltpu.SemaphoreType.DMA((NBUF,)),
            pltpu.SemaphoreType.DMA((NBUF,)),
        ],
    )
    return k(t1, er1, src_r, dst_r, z80)


def _sc_layer2_body(t2_hbm, er_hbm, src_hbm, dst_hbm, z_hbm, out_hbm,
                    srcv, dstv, t2b, erb, msgb, acc,
                    gsem1, gsem2, ssem):
    cid = lax.axis_index("c")
    sid = lax.axis_index("s")
    wid = cid * 16 + sid
    _rowcopy(z_hbm, acc, sid)
    pltpu.sync_copy(src_hbm.at[wid], srcv)
    pltpu.sync_copy(dst_hbm.at[wid], dstv)
    plsc.subcore_barrier()

    il = lax.iota(jnp.int32, 16)
    zero = jnp.zeros((16,), jnp.float32)
    full8 = jnp.full((16,), 8, jnp.int32)
    full0 = jnp.zeros((16,), jnp.int32)

    def gathers(c, b):
        pltpu.async_copy(t2_hbm.at[srcv.at[c]], t2b.at[b], gsem1.at[b])
        pltpu.async_copy(er_hbm.at[dstv.at[c]], erb.at[b], gsem2.at[b])

    for b in range(NBUF):
        gathers(b, b)

    @pl.loop(0, NCH, step=NBUF)
    def _chunk(j):
        for b in range(NBUF):
            c = j + b
            pltpu.make_async_copy(t2_hbm.at[srcv.at[c]],
                                  t2b.at[b], gsem1.at[b]).wait()
            pltpu.make_async_copy(er_hbm.at[dstv.at[c]],
                                  erb.at[b], gsem2.at[b]).wait()

            @pl.when(c >= NBUF)
            def _():
                pltpu.make_async_copy(msgb.at[b], acc.at[dstv.at[c]],
                                      ssem.at[b]).wait()

            @plsc.parallel_loop(0, CHUNK, unroll=4)
            def _edge(e):
                erv = erb[b, e, :]
                m0, m1 = _unpack32(t2b[b, e, pl.ds(0, 32)])
                m2, _ = _unpack32(t2b[b, e, pl.ds(32, 32)])
                ev = _vgather(m2, full8) + _vgather(erv, full0)
                ev = jnp.where(ev >= 0.0, ev, 0.2 * ev)
                sb = jnp.exp(ev)
                msgb[b, e, pl.ds(0, 16)] = m0 * sb
                msgb[b, e, pl.ds(16, 16)] = m1 * sb
                mv = jnp.where(il < 8, m2 * sb,
                               jnp.where(il == 8, sb, zero))
                msgb[b, e, pl.ds(32, 16)] = mv

            pltpu.async_copy(msgb.at[b], acc.at[dstv.at[c]], ssem.at[b],
                             add=True)

            @pl.when(c + NBUF < NCH)
            def _():
                gathers(c + NBUF, b)

    for b in range(NBUF):
        pltpu.make_async_copy(msgb.at[b], acc.at[pl.ds(0, CHUNK)],
                              ssem.at[b]).wait()

    plsc.subcore_barrier()
    _rowcopy(acc, out_hbm.at[cid], sid)


def _sc_layer2(t2, er2, src_r, dst_r, z48):
    k = pl.kernel(
        _sc_layer2_body,
        out_type=jax.ShapeDtypeStruct((2, N, TW2), jnp.float32),
        mesh=_mesh(),
        compiler_params=_sc_params(),
        scratch_types=[
            pltpu.VMEM((NCH, CHUNK), jnp.int32),
            pltpu.VMEM((NCH, CHUNK), jnp.int32),
            pltpu.VMEM((NBUF, CHUNK, GW2), jnp.bfloat16),
            pltpu.VMEM((NBUF, CHUNK, ERW), jnp.float32),
            pltpu.VMEM((NBUF, CHUNK, TW2), jnp.float32),
            pltpu.VMEM_SHARED((N, TW2), jnp.float32),
            pltpu.SemaphoreType.DMA((NBUF,)),
            pltpu.SemaphoreType.DMA((NBUF,)),
            pltpu.SemaphoreType.DMA((NBUF,)),
        ],
    )
    return k(t2, er2, src_r, dst_r, z48)


def kernel(x, edge_index, W1, al1, ar1, b1, W2, al2, ar2, b2):
    w1r = W1.reshape(H1, F1, F_IN)
    a_l1 = jnp.einsum("hfk,hf->kh", w1r, al1[0])
    a_r1 = jnp.einsum("hfk,hf->kh", w1r, ar1[0])
    t1_nat = jnp.concatenate(
        [W1.T, a_l1, jnp.zeros((F_IN, 24), jnp.float32)], axis=1)
    wc1 = jnp.concatenate(
        [t1_nat[:, _bf16_perm(GW1)], a_r1,
         jnp.zeros((F_IN, 8), jnp.float32)], axis=1)

    w2r = W2.reshape(H2, F2, D1)
    a_l2 = jnp.einsum("hfk,hf->kh", w2r, al2[0])
    a_r2 = jnp.einsum("hfk,hf->kh", w2r, ar2[0])
    t2_nat = jnp.concatenate(
        [W2.T, a_l2, jnp.zeros((D1, 23), jnp.float32)], axis=1)
    wc2 = jnp.concatenate(
        [t2_nat[:, _bf16_perm(GW2)], a_r2,
         jnp.zeros((D1, 15), jnp.float32)], axis=1)

    src_r = edge_index[0].reshape(NTILES, NCH, CHUNK)
    dst_r = edge_index[1].reshape(NTILES, NCH, CHUNK)
    z80 = jnp.zeros((N, TW1), jnp.float32)
    z48 = jnp.zeros((N, TW2), jnp.float32)
    b1row = b1.reshape(1, D1)
    b2row = b2.reshape(1, F2)

    t1, er1 = _tc_project(x, wc1, GW1)
    p1 = _sc_layer1(t1, er1, src_r, dst_r, z80)
    t2, er2 = _tc_mid(p1, wc2, b1row)
    p2 = _sc_layer2(t2, er2, src_r, dst_r, z48)
    return _tc_final(p2, b2row)


# speedup vs baseline: 186.4578x; 1.0201x over previous
import dataclasses
import functools

import jax
import jax.numpy as jnp
from jax import lax
from jax.experimental import pallas as pl
from jax.experimental.pallas import tpu as pltpu
from jax.experimental.pallas import tpu_sc as plsc

N = 10000
E = 320000
F_IN = 128
H1, F1 = 8, 8
D1 = H1 * F1
H2, F2 = 1, 40
D2 = H2 * F2

TW1 = 80
TW2 = 48
GW1 = 96
GW2 = 64
ERW = 16

NTILES = 32
CHUNK = 125
EPT = E // NTILES
NCH = EPT // CHUNK
NBUF = 2
R0 = 624
RLAST = N - 15 * R0

def _bf16_perm(width):
    p = []
    for blk in range(width // 32):
        for i in range(16):
            p.append(32 * blk + i)
            p.append(32 * blk + 16 + i)
    return p


def _unpack32(v):
    return plsc.unpack(v, format=plsc.PackFormat.INTERLEAVED,
                       preferred_element_type=jnp.float32)


def _vgather(x, idx):
    dnums = lax.GatherDimensionNumbers(
        offset_dims=(), collapsed_slice_dims=(0,), start_index_map=(0,))
    return lax.gather(x, idx[:, None], dnums, (1,),
                      mode=lax.GatherScatterMode.PROMISE_IN_BOUNDS)


def _mesh():
    return plsc.VectorSubcoreMesh(core_axis_name="c", subcore_axis_name="s")


def _sc_params():
    cp = pltpu.CompilerParams()
    fields = pltpu.CompilerParams.__dataclass_fields__
    if "needs_layout_passes" in fields:
        cp = dataclasses.replace(cp, needs_layout_passes=False)
    if "use_tc_tiling_on_sc" in fields:
        cp = dataclasses.replace(cp, use_tc_tiling_on_sc=False)
    return cp


def _mm_body(x_ref, w_ref, a_ref, b_ref, split):
    y = jnp.dot(x_ref[...], w_ref[...], preferred_element_type=jnp.float32)
    a_ref[...] = y[:, :split].astype(a_ref.dtype)
    b_ref[...] = y[:, split:]


def _tc_project(x, w, split, rows_per_blk=2000):
    n, k = x.shape
    m = w.shape[1]
    grid = (n // rows_per_blk,)
    return pl.pallas_call(
        functools.partial(_mm_body, split=split),
        grid=grid,
        in_specs=[
            pl.BlockSpec((rows_per_blk, k), lambda i: (i, 0)),
            pl.BlockSpec((k, m), lambda i: (0, 0)),
        ],
        out_specs=[
            pl.BlockSpec((rows_per_blk, split), lambda i: (i, 0)),
            pl.BlockSpec((rows_per_blk, m - split), lambda i: (i, 0)),
        ],
        out_shape=[
            jax.ShapeDtypeStruct((n, split), jnp.bfloat16),
            jax.ShapeDtypeStruct((n, m - split), jnp.float32),
        ],
    )(x, w)


def _mid_body(p_ref, w_ref, b_ref, t2_ref, er_ref):
    a = p_ref[0] + p_ref[1]
    d = a[:, D1:D1 + H1]
    r = 1.0 / jnp.where(d != 0.0, d, 1.0)
    pieces = []
    for h in range(H1):
        pieces.append(a[:, F1 * h:F1 * h + F1] * r[:, h:h + 1])
    o = jnp.concatenate(pieces, axis=1)
    hb = jnp.maximum(o + b_ref[...], 0.0)
    y = jnp.dot(hb, w_ref[...], preferred_element_type=jnp.float32)
    t2_ref[...] = y[:, :GW2].astype(jnp.bfloat16)
    er_ref[...] = y[:, GW2:]


def _tc_mid(p1, w, b1row, rows_per_blk=2000):
    grid = (N // rows_per_blk,)
    return pl.pallas_call(
        _mid_body,
        grid=grid,
        in_specs=[
            pl.BlockSpec((2, rows_per_blk, TW1), lambda i: (0, i, 0)),
            pl.BlockSpec(w.shape, lambda i: (0, 0)),
            pl.BlockSpec((1, D1), lambda i: (0, 0)),
        ],
        out_specs=[
            pl.BlockSpec((rows_per_blk, GW2), lambda i: (i, 0)),
            pl.BlockSpec((rows_per_blk, ERW), lambda i: (i, 0)),
        ],
        out_shape=[
            jax.ShapeDtypeStruct((N, GW2), jnp.bfloat16),
            jax.ShapeDtypeStruct((N, ERW), jnp.float32),
        ],
    )(p1, w, b1row)


def _final_body(p_ref, b_ref, o_ref):
    a = p_ref[0] + p_ref[1]
    d = a[:, F2:F2 + 1]
    d = jnp.where(d != 0.0, d, 1.0)
    z = a[:, :F2] / d + b_ref[...]
    m = jnp.max(z, axis=1, keepdims=True)
    lse = m + jnp.log(jnp.sum(jnp.exp(z - m), axis=1, keepdims=True))
    o_ref[...] = z - lse


def _tc_final(p2, b2row, rows_per_blk=2000):
    grid = (N // rows_per_blk,)
    return pl.pallas_call(
        _final_body,
        grid=grid,
        in_specs=[
            pl.BlockSpec((2, rows_per_blk, TW2), lambda i: (0, i, 0)),
            pl.BlockSpec((1, F2), lambda i: (0, 0)),
        ],
        out_specs=pl.BlockSpec((rows_per_blk, F2), lambda i: (i, 0)),
        out_shape=jax.ShapeDtypeStruct((N, F2), jnp.float32),
    )(p2, b2row)


def _sc_layer1_body(t1_hbm, er_hbm, src_hbm, dst_hbm, z_hbm, out_hbm,
                    srcv, dstv, t1b, erb, msgb, acc,
                    gsem1, gsem2, ssem):
    cid = lax.axis_index("c")
    sid = lax.axis_index("s")
    wid = cid * 16 + sid
    _rowcopy(z_hbm, acc, sid)
    pltpu.sync_copy(src_hbm.at[wid], srcv)
    pltpu.sync_copy(dst_hbm.at[wid], dstv)
    plsc.subcore_barrier()

    il = lax.iota(jnp.int32, 16)
    head_sel = il >> 3
    takeidx = [2 * q + head_sel for q in range(4)]

    def gathers(c, b):
        pltpu.async_copy(t1_hbm.at[srcv.at[c]], t1b.at[b], gsem1.at[b])
        pltpu.async_copy(er_hbm.at[dstv.at[c]], erb.at[b], gsem2.at[b])

    for b in range(NBUF):
        gathers(b, b)

    @pl.loop(0, NCH, step=NBUF)
    def _chunk(j):
        for b in range(NBUF):
            c = j + b
            pltpu.make_async_copy(t1_hbm.at[srcv.at[c]],
                                  t1b.at[b], gsem1.at[b]).wait()
            pltpu.make_async_copy(er_hbm.at[dstv.at[c]],
                                  erb.at[b], gsem2.at[b]).wait()

            @pl.when(c >= NBUF)
            def _():
                pltpu.make_async_copy(msgb.at[b], acc.at[dstv.at[c]],
                                      ssem.at[b]).wait()

            @plsc.parallel_loop(0, CHUNK, unroll=4)
            def _edge(e):
                elv, _ = _unpack32(t1b[b, e, pl.ds(D1, 32)])
                erv = erb[b, e, :]
                ev = elv + erv
                ev = jnp.where(ev >= 0.0, ev, 0.2 * ev)
                sv = jnp.exp(ev)
                msgb[b, e, pl.ds(D1, 16)] = sv
                for q2 in range(2):
                    ha, hbv = _unpack32(t1b[b, e, pl.ds(32 * q2, 32)])
                    msgb[b, e, pl.ds(32 * q2, 16)] = (
                        ha * _vgather(sv, takeidx[2 * q2]))
                    msgb[b, e, pl.ds(32 * q2 + 16, 16)] = (
                        hbv * _vgather(sv, takeidx[2 * q2 + 1]))

            pltpu.async_copy(msgb.at[b], acc.at[dstv.at[c]], ssem.at[b],
                             add=True)

            @pl.when(c + NBUF < NCH)
            def _():
                gathers(c + NBUF, b)

    for b in range(NBUF):
        pltpu.make_async_copy(msgb.at[b], acc.at[pl.ds(0, CHUNK)],
                              ssem.at[b]).wait()

    plsc.subcore_barrier()
    _rowcopy(acc, out_hbm.at[cid], sid)


def _rowcopy(src, dst, sid):
    @pl.when(sid < 15)
    def _():
        st = pl.multiple_of(sid * R0, 8)
        pltpu.sync_copy(src.at[pl.ds(st, R0)], dst.at[pl.ds(st, R0)])

    @pl.when(sid == 15)
    def _():
        pltpu.sync_copy(src.at[pl.ds(15 * R0, RLAST)],
                        dst.at[pl.ds(15 * R0, RLAST)])


def _sc_layer1(t1, er1, src_r, dst_r, z80):
    k = pl.kernel(
        _sc_layer1_body,
        out_type=jax.ShapeDtypeStruct((2, N, TW1), jnp.float32),
        mesh=_mesh(),
        compiler_params=_sc_params(),
        scratch_types=[
            pltpu.VMEM((NCH, CHUNK), jnp.int32),
            pltpu.VMEM((NCH, CHUNK), jnp.int32),
            pltpu.VMEM((NBUF, CHUNK, GW1), jnp.bfloat16),
            pltpu.VMEM((NBUF, CHUNK, ERW), jnp.float32),
            pltpu.VMEM((NBUF, CHUNK, TW1), jnp.float32),
            pltpu.VMEM_SHARED((N, TW1), jnp.float32),
            pltpu.SemaphoreType.DMA((NBUF,)),
            pltpu.SemaphoreType.DMA((NBUF,)),
            pltpu.SemaphoreType.DMA((NBUF,)),
        ],
    )
    return k(t1, er1, src_r, dst_r, z80)


def _sc_layer2_body(t2_hbm, er_hbm, src_hbm, dst_hbm, z_hbm, out_hbm,
                    srcv, dstv, t2b, erb, msgb, acc,
                    gsem1, gsem2, ssem):
    cid = lax.axis_index("c")
    sid = lax.axis_index("s")
    wid = cid * 16 + sid
    _rowcopy(z_hbm, acc, sid)
    pltpu.sync_copy(src_hbm.at[wid], srcv)
    pltpu.sync_copy(dst_hbm.at[wid], dstv)
    plsc.subcore_barrier()

    il = lax.iota(jnp.int32, 16)
    zero = jnp.zeros((16,), jnp.float32)
    full8 = jnp.full((16,), 8, jnp.int32)
    full0 = jnp.zeros((16,), jnp.int32)

    def gathers(c, b):
        pltpu.async_copy(t2_hbm.at[srcv.at[c]], t2b.at[b], gsem1.at[b])
        pltpu.async_copy(er_hbm.at[dstv.at[c]], erb.at[b], gsem2.at[b])

    for b in range(NBUF):
        gathers(b, b)

    @pl.loop(0, NCH, step=NBUF)
    def _chunk(j):
        for b in range(NBUF):
            c = j + b
            pltpu.make_async_copy(t2_hbm.at[srcv.at[c]],
                                  t2b.at[b], gsem1.at[b]).wait()
            pltpu.make_async_copy(er_hbm.at[dstv.at[c]],
                                  erb.at[b], gsem2.at[b]).wait()

            @pl.when(c >= NBUF)
            def _():
                pltpu.make_async_copy(msgb.at[b], acc.at[dstv.at[c]],
                                      ssem.at[b]).wait()

            @plsc.parallel_loop(0, CHUNK, unroll=4)
            def _edge(e):
                erv = erb[b, e, :]
                m0, m1 = _unpack32(t2b[b, e, pl.ds(0, 32)])
                m2, _ = _unpack32(t2b[b, e, pl.ds(32, 32)])
                ev = _vgather(m2, full8) + _vgather(erv, full0)
                ev = jnp.where(ev >= 0.0, ev, 0.2 * ev)
                sb = jnp.exp(ev)
                msgb[b, e, pl.ds(0, 16)] = m0 * sb
                msgb[b, e, pl.ds(16, 16)] = m1 * sb
                mv = jnp.where(il < 8, m2 * sb,
                               jnp.where(il == 8, sb, zero))
                msgb[b, e, pl.ds(32, 16)] = mv

            pltpu.async_copy(msgb.at[b], acc.at[dstv.at[c]], ssem.at[b],
                             add=True)

            @pl.when(c + NBUF < NCH)
            def _():
                gathers(c + NBUF, b)

    for b in range(NBUF):
        pltpu.make_async_copy(msgb.at[b], acc.at[pl.ds(0, CHUNK)],
                              ssem.at[b]).wait()

    plsc.subcore_barrier()
    _rowcopy(acc, out_hbm.at[cid], sid)


def _sc_layer2(t2, er2, src_r, dst_r, z48):
    k = pl.kernel(
        _sc_layer2_body,
        out_type=jax.ShapeDtypeStruct((2, N, TW2), jnp.float32),
        mesh=_mesh(),
        compiler_params=_sc_params(),
        scratch_types=[
            pltpu.VMEM((NCH, CHUNK), jnp.int32),
            pltpu.VMEM((NCH, CHUNK), jnp.int32),
            pltpu.VMEM((NBUF, CHUNK, GW2), jnp.bfloat16),
            pltpu.VMEM((NBUF, CHUNK, ERW), jnp.float32),
            pltpu.VMEM((NBUF, CHUNK, TW2), jnp.float32),
            pltpu.VMEM_SHARED((N, TW2), jnp.float32),
            pltpu.SemaphoreType.DMA((NBUF,)),
            pltpu.SemaphoreType.DMA((NBUF,)),
            pltpu.SemaphoreType.DMA((NBUF,)),
        ],
    )
    return k(t2, er2, src_r, dst_r, z48)


def kernel(x, edge_index, W1, al1, ar1, b1, W2, al2, ar2, b2):
    w1r = W1.reshape(H1, F1, F_IN)
    a_l1 = jnp.einsum("hfk,hf->kh", w1r, al1[0])
    a_r1 = jnp.einsum("hfk,hf->kh", w1r, ar1[0])
    t1_nat = jnp.concatenate(
        [W1.T, a_l1, jnp.zeros((F_IN, 24), jnp.float32)], axis=1)
    wc1 = jnp.concatenate(
        [t1_nat[:, _bf16_perm(GW1)], a_r1,
         jnp.zeros((F_IN, 8), jnp.float32)], axis=1)

    w2r = W2.reshape(H2, F2, D1)
    a_l2 = jnp.einsum("hfk,hf->kh", w2r, al2[0])
    a_r2 = jnp.einsum("hfk,hf->kh", w2r, ar2[0])
    t2_nat = jnp.concatenate(
        [W2.T, a_l2, jnp.zeros((D1, 23), jnp.float32)], axis=1)
    wc2 = jnp.concatenate(
        [t2_nat[:, _bf16_perm(GW2)], a_r2,
         jnp.zeros((D1, 15), jnp.float32)], axis=1)

    src_r = edge_index[0].reshape(NTILES, NCH, CHUNK)
    dst_r = edge_index[1].reshape(NTILES, NCH, CHUNK)
    z80 = jnp.zeros((N, TW1), jnp.float32)
    z48 = jnp.zeros((N, TW2), jnp.float32)
    b1row = b1.reshape(1, D1)
    b2row = b2.reshape(1, F2)

    t1, er1 = _tc_project(x, wc1, GW1)
    p1 = _sc_layer1(t1, er1, src_r, dst_r, z80)
    t2, er2 = _tc_mid(p1, wc2, b1row)
    p2 = _sc_layer2(t2, er2, src_r, dst_r, z48)
    return _tc_final(p2, b2row)


# speedup vs baseline: 198.4101x; 1.0641x over previous
import dataclasses
import functools

import jax
import jax.numpy as jnp
from jax import lax
from jax.experimental import pallas as pl
from jax.experimental.pallas import tpu as pltpu
from jax.experimental.pallas import tpu_sc as plsc

N = 10000
E = 320000
F_IN = 128
H1, F1 = 8, 8
D1 = H1 * F1
H2, F2 = 1, 40
D2 = H2 * F2

TW1 = 80
TW2 = 48
GW1 = 96
GW2 = 64
ERW = 16

NTILES = 32
CHUNK = 100
EPT = E // NTILES
NCH = EPT // CHUNK
NBUF = 4
R0 = 624
RLAST = N - 15 * R0

def _bf16_perm(width):
    p = []
    for blk in range(width // 32):
        for i in range(16):
            p.append(32 * blk + i)
            p.append(32 * blk + 16 + i)
    return p


def _unpack32(v):
    return plsc.unpack(v, format=plsc.PackFormat.INTERLEAVED,
                       preferred_element_type=jnp.float32)


def _vgather(x, idx):
    dnums = lax.GatherDimensionNumbers(
        offset_dims=(), collapsed_slice_dims=(0,), start_index_map=(0,))
    return lax.gather(x, idx[:, None], dnums, (1,),
                      mode=lax.GatherScatterMode.PROMISE_IN_BOUNDS)


def _mesh():
    return plsc.VectorSubcoreMesh(core_axis_name="c", subcore_axis_name="s")


def _sc_params():
    cp = pltpu.CompilerParams()
    fields = pltpu.CompilerParams.__dataclass_fields__
    if "needs_layout_passes" in fields:
        cp = dataclasses.replace(cp, needs_layout_passes=False)
    if "use_tc_tiling_on_sc" in fields:
        cp = dataclasses.replace(cp, use_tc_tiling_on_sc=False)
    return cp


def _mm_body(x_ref, w_ref, a_ref, b_ref, split):
    y = jnp.dot(x_ref[...], w_ref[...], preferred_element_type=jnp.float32)
    a_ref[...] = y[:, :split].astype(a_ref.dtype)
    b_ref[...] = y[:, split:]


def _tc_project(x, w, split, rows_per_blk=2000):
    n, k = x.shape
    m = w.shape[1]
    grid = (n // rows_per_blk,)
    return pl.pallas_call(
        functools.partial(_mm_body, split=split),
        grid=grid,
        in_specs=[
            pl.BlockSpec((rows_per_blk, k), lambda i: (i, 0)),
            pl.BlockSpec((k, m), lambda i: (0, 0)),
        ],
        out_specs=[
            pl.BlockSpec((rows_per_blk, split), lambda i: (i, 0)),
            pl.BlockSpec((rows_per_blk, m - split), lambda i: (i, 0)),
        ],
        out_shape=[
            jax.ShapeDtypeStruct((n, split), jnp.bfloat16),
            jax.ShapeDtypeStruct((n, m - split), jnp.float32),
        ],
    )(x, w)


def _mid_body(p_ref, w_ref, b_ref, t2_ref, er_ref):
    a = p_ref[0] + p_ref[1]
    d = a[:, D1:D1 + H1]
    r = 1.0 / jnp.where(d != 0.0, d, 1.0)
    pieces = []
    for h in range(H1):
        pieces.append(a[:, F1 * h:F1 * h + F1] * r[:, h:h + 1])
    o = jnp.concatenate(pieces, axis=1)
    hb = jnp.maximum(o + b_ref[...], 0.0)
    y = jnp.dot(hb, w_ref[...], preferred_element_type=jnp.float32)
    t2_ref[...] = y[:, :GW2].astype(jnp.bfloat16)
    er_ref[...] = y[:, GW2:]


def _tc_mid(p1, w, b1row, rows_per_blk=2000):
    grid = (N // rows_per_blk,)
    return pl.pallas_call(
        _mid_body,
        grid=grid,
        in_specs=[
            pl.BlockSpec((2, rows_per_blk, TW1), lambda i: (0, i, 0)),
            pl.BlockSpec(w.shape, lambda i: (0, 0)),
            pl.BlockSpec((1, D1), lambda i: (0, 0)),
        ],
        out_specs=[
            pl.BlockSpec((rows_per_blk, GW2), lambda i: (i, 0)),
            pl.BlockSpec((rows_per_blk, ERW), lambda i: (i, 0)),
        ],
        out_shape=[
            jax.ShapeDtypeStruct((N, GW2), jnp.bfloat16),
            jax.ShapeDtypeStruct((N, ERW), jnp.float32),
        ],
    )(p1, w, b1row)


def _final_body(p_ref, b_ref, o_ref):
    a = p_ref[0] + p_ref[1]
    d = a[:, F2:F2 + 1]
    d = jnp.where(d != 0.0, d, 1.0)
    z = a[:, :F2] / d + b_ref[...]
    m = jnp.max(z, axis=1, keepdims=True)
    lse = m + jnp.log(jnp.sum(jnp.exp(z - m), axis=1, keepdims=True))
    o_ref[...] = z - lse


def _tc_final(p2, b2row, rows_per_blk=2000):
    grid = (N // rows_per_blk,)
    return pl.pallas_call(
        _final_body,
        grid=grid,
        in_specs=[
            pl.BlockSpec((2, rows_per_blk, TW2), lambda i: (0, i, 0)),
            pl.BlockSpec((1, F2), lambda i: (0, 0)),
        ],
        out_specs=pl.BlockSpec((rows_per_blk, F2), lambda i: (i, 0)),
        out_shape=jax.ShapeDtypeStruct((N, F2), jnp.float32),
    )(p2, b2row)


def _sc_layer1_body(t1_hbm, er_hbm, src_hbm, dst_hbm, z_hbm, out_hbm,
                    srcv, dstv, t1b, erb, msgb, acc,
                    gsem1, gsem2, ssem):
    cid = lax.axis_index("c")
    sid = lax.axis_index("s")
    wid = cid * 16 + sid
    _rowcopy(z_hbm, acc, sid)
    pltpu.sync_copy(src_hbm.at[wid], srcv)
    pltpu.sync_copy(dst_hbm.at[wid], dstv)
    plsc.subcore_barrier()

    il = lax.iota(jnp.int32, 16)
    head_sel = il >> 3
    takeidx = [2 * q + head_sel for q in range(4)]

    def gathers(c, b):
        pltpu.async_copy(t1_hbm.at[srcv.at[c]], t1b.at[b], gsem1.at[b])
        pltpu.async_copy(er_hbm.at[dstv.at[c]], erb.at[b], gsem2.at[b])

    for b in range(NBUF):
        gathers(b, b)

    @pl.loop(0, NCH, step=NBUF)
    def _chunk(j):
        for b in range(NBUF):
            c = j + b
            pltpu.make_async_copy(t1_hbm.at[srcv.at[c]],
                                  t1b.at[b], gsem1.at[b]).wait()
            pltpu.make_async_copy(er_hbm.at[dstv.at[c]],
                                  erb.at[b], gsem2.at[b]).wait()

            @pl.when(c >= NBUF)
            def _():
                pltpu.make_async_copy(msgb.at[b], acc.at[dstv.at[c]],
                                      ssem.at[b]).wait()

            @plsc.parallel_loop(0, CHUNK, unroll=4)
            def _edge(e):
                elv, _ = _unpack32(t1b[b, e, pl.ds(D1, 32)])
                erv = erb[b, e, :]
                ev = elv + erv
                ev = jnp.where(ev >= 0.0, ev, 0.2 * ev)
                sv = jnp.exp(ev)
                msgb[b, e, pl.ds(D1, 16)] = sv
                for q2 in range(2):
                    ha, hbv = _unpack32(t1b[b, e, pl.ds(32 * q2, 32)])
                    msgb[b, e, pl.ds(32 * q2, 16)] = (
                        ha * _vgather(sv, takeidx[2 * q2]))
                    msgb[b, e, pl.ds(32 * q2 + 16, 16)] = (
                        hbv * _vgather(sv, takeidx[2 * q2 + 1]))

            pltpu.async_copy(msgb.at[b], acc.at[dstv.at[c]], ssem.at[b],
                             add=True)

            @pl.when(c + NBUF < NCH)
            def _():
                gathers(c + NBUF, b)

    for b in range(NBUF):
        pltpu.make_async_copy(msgb.at[b], acc.at[pl.ds(0, CHUNK)],
                              ssem.at[b]).wait()

    plsc.subcore_barrier()
    _rowcopy(acc, out_hbm.at[cid], sid)


def _rowcopy(src, dst, sid):
    @pl.when(sid < 15)
    def _():
        st = pl.multiple_of(sid * R0, 8)
        pltpu.sync_copy(src.at[pl.ds(st, R0)], dst.at[pl.ds(st, R0)])

    @pl.when(sid == 15)
    def _():
        pltpu.sync_copy(src.at[pl.ds(15 * R0, RLAST)],
                        dst.at[pl.ds(15 * R0, RLAST)])


def _sc_layer1(t1, er1, src_r, dst_r, z80):
    k = pl.kernel(
        _sc_layer1_body,
        out_type=jax.ShapeDtypeStruct((2, N, TW1), jnp.float32),
        mesh=_mesh(),
        compiler_params=_sc_params(),
        scratch_types=[
            pltpu.VMEM((NCH, CHUNK), jnp.int32),
            pltpu.VMEM((NCH, CHUNK), jnp.int32),
            pltpu.VMEM((NBUF, CHUNK, GW1), jnp.bfloat16),
            pltpu.VMEM((NBUF, CHUNK, ERW), jnp.float32),
            pltpu.VMEM((NBUF, CHUNK, TW1), jnp.float32),
            pltpu.VMEM_SHARED((N, TW1), jnp.float32),
            pltpu.SemaphoreType.DMA((NBUF,)),
            pltpu.SemaphoreType.DMA((NBUF,)),
            pltpu.SemaphoreType.DMA((NBUF,)),
        ],
    )
    return k(t1, er1, src_r, dst_r, z80)


def _sc_layer2_body(t2_hbm, er_hbm, src_hbm, dst_hbm, z_hbm, out_hbm,
                    srcv, dstv, t2b, erb, msgb, acc,
                    gsem1, gsem2, ssem):
    cid = lax.axis_index("c")
    sid = lax.axis_index("s")
    wid = cid * 16 + sid
    _rowcopy(z_hbm, acc, sid)
    pltpu.sync_copy(src_hbm.at[wid], srcv)
    pltpu.sync_copy(dst_hbm.at[wid], dstv)
    plsc.subcore_barrier()

    il = lax.iota(jnp.int32, 16)
    zero = jnp.zeros((16,), jnp.float32)
    full8 = jnp.full((16,), 8, jnp.int32)
    full0 = jnp.zeros((16,), jnp.int32)

    def gathers(c, b):
        pltpu.async_copy(t2_hbm.at[srcv.at[c]], t2b.at[b], gsem1.at[b])
        pltpu.async_copy(er_hbm.at[dstv.at[c]], erb.at[b], gsem2.at[b])

    for b in range(NBUF):
        gathers(b, b)

    @pl.loop(0, NCH, step=NBUF)
    def _chunk(j):
        for b in range(NBUF):
            c = j + b
            pltpu.make_async_copy(t2_hbm.at[srcv.at[c]],
                                  t2b.at[b], gsem1.at[b]).wait()
            pltpu.make_async_copy(er_hbm.at[dstv.at[c]],
                                  erb.at[b], gsem2.at[b]).wait()

            @pl.when(c >= NBUF)
            def _():
                pltpu.make_async_copy(msgb.at[b], acc.at[dstv.at[c]],
                                      ssem.at[b]).wait()

            @plsc.parallel_loop(0, CHUNK, unroll=4)
            def _edge(e):
                erv = erb[b, e, :]
                m0, m1 = _unpack32(t2b[b, e, pl.ds(0, 32)])
                m2, _ = _unpack32(t2b[b, e, pl.ds(32, 32)])
                ev = _vgather(m2, full8) + _vgather(erv, full0)
                ev = jnp.where(ev >= 0.0, ev, 0.2 * ev)
                sb = jnp.exp(ev)
                msgb[b, e, pl.ds(0, 16)] = m0 * sb
                msgb[b, e, pl.ds(16, 16)] = m1 * sb
                mv = jnp.where(il < 8, m2 * sb,
                               jnp.where(il == 8, sb, zero))
                msgb[b, e, pl.ds(32, 16)] = mv

            pltpu.async_copy(msgb.at[b], acc.at[dstv.at[c]], ssem.at[b],
                             add=True)

            @pl.when(c + NBUF < NCH)
            def _():
                gathers(c + NBUF, b)

    for b in range(NBUF):
        pltpu.make_async_copy(msgb.at[b], acc.at[pl.ds(0, CHUNK)],
                              ssem.at[b]).wait()

    plsc.subcore_barrier()
    _rowcopy(acc, out_hbm.at[cid], sid)


def _sc_layer2(t2, er2, src_r, dst_r, z48):
    k = pl.kernel(
        _sc_layer2_body,
        out_type=jax.ShapeDtypeStruct((2, N, TW2), jnp.float32),
        mesh=_mesh(),
        compiler_params=_sc_params(),
        scratch_types=[
            pltpu.VMEM((NCH, CHUNK), jnp.int32),
            pltpu.VMEM((NCH, CHUNK), jnp.int32),
            pltpu.VMEM((NBUF, CHUNK, GW2), jnp.bfloat16),
            pltpu.VMEM((NBUF, CHUNK, ERW), jnp.float32),
            pltpu.VMEM((NBUF, CHUNK, TW2), jnp.float32),
            pltpu.VMEM_SHARED((N, TW2), jnp.float32),
            pltpu.SemaphoreType.DMA((NBUF,)),
            pltpu.SemaphoreType.DMA((NBUF,)),
            pltpu.SemaphoreType.DMA((NBUF,)),
        ],
    )
    return k(t2, er2, src_r, dst_r, z48)


def kernel(x, edge_index, W1, al1, ar1, b1, W2, al2, ar2, b2):
    w1r = W1.reshape(H1, F1, F_IN)
    a_l1 = jnp.einsum("hfk,hf->kh", w1r, al1[0])
    a_r1 = jnp.einsum("hfk,hf->kh", w1r, ar1[0])
    t1_nat = jnp.concatenate(
        [W1.T, a_l1, jnp.zeros((F_IN, 24), jnp.float32)], axis=1)
    wc1 = jnp.concatenate(
        [t1_nat[:, _bf16_perm(GW1)], a_r1,
         jnp.zeros((F_IN, 8), jnp.float32)], axis=1)

    w2r = W2.reshape(H2, F2, D1)
    a_l2 = jnp.einsum("hfk,hf->kh", w2r, al2[0])
    a_r2 = jnp.einsum("hfk,hf->kh", w2r, ar2[0])
    t2_nat = jnp.concatenate(
        [W2.T, a_l2, jnp.zeros((D1, 23), jnp.float32)], axis=1)
    wc2 = jnp.concatenate(
        [t2_nat[:, _bf16_perm(GW2)], a_r2,
         jnp.zeros((D1, 15), jnp.float32)], axis=1)

    src_r = edge_index[0].reshape(NTILES, NCH, CHUNK)
    dst_r = edge_index[1].reshape(NTILES, NCH, CHUNK)
    z80 = jnp.zeros((N, TW1), jnp.float32)
    z48 = jnp.zeros((N, TW2), jnp.float32)
    b1row = b1.reshape(1, D1)
    b2row = b2.reshape(1, F2)

    t1, er1 = _tc_project(x, wc1, GW1)
    p1 = _sc_layer1(t1, er1, src_r, dst_r, z80)
    t2, er2 = _tc_mid(p1, wc2, b1row)
    p2 = _sc_layer2(t2, er2, src_r, dst_r, z48)
    return _tc_final(p2, b2row)


# speedup vs baseline: 198.4793x; 1.0003x over previous
import dataclasses
import functools

import jax
import jax.numpy as jnp
from jax import lax
from jax.experimental import pallas as pl
from jax.experimental.pallas import tpu as pltpu
from jax.experimental.pallas import tpu_sc as plsc

N = 10000
E = 320000
F_IN = 128
H1, F1 = 8, 8
D1 = H1 * F1
H2, F2 = 1, 40
D2 = H2 * F2

TW1 = 80
TW2 = 48
GW1 = 96
GW2 = 64
ERW = 16

NTILES = 32
CHUNK = 100
EPT = E // NTILES
NCH = EPT // CHUNK
NBUF = 4
R0 = 624
RLAST = N - 15 * R0

def _bf16_perm(width):
    p = []
    for blk in range(width // 32):
        for i in range(16):
            p.append(32 * blk + i)
            p.append(32 * blk + 16 + i)
    return p


def _unpack32(v):
    return plsc.unpack(v, format=plsc.PackFormat.INTERLEAVED,
                       preferred_element_type=jnp.float32)


def _vgather(x, idx):
    dnums = lax.GatherDimensionNumbers(
        offset_dims=(), collapsed_slice_dims=(0,), start_index_map=(0,))
    return lax.gather(x, idx[:, None], dnums, (1,),
                      mode=lax.GatherScatterMode.PROMISE_IN_BOUNDS)


def _mesh():
    return plsc.VectorSubcoreMesh(core_axis_name="c", subcore_axis_name="s")


def _sc_params():
    cp = pltpu.CompilerParams()
    fields = pltpu.CompilerParams.__dataclass_fields__
    if "needs_layout_passes" in fields:
        cp = dataclasses.replace(cp, needs_layout_passes=False)
    if "use_tc_tiling_on_sc" in fields:
        cp = dataclasses.replace(cp, use_tc_tiling_on_sc=False)
    return cp


def _mm_body(x_ref, w_ref, a_ref, b_ref, split):
    y = jnp.dot(x_ref[...], w_ref[...], preferred_element_type=jnp.float32)
    a_ref[...] = y[:, :split].astype(a_ref.dtype)
    b_ref[...] = y[:, split:]


def _tc_project(x, w, split, rows_per_blk=2000):
    n, k = x.shape
    m = w.shape[1]
    grid = (n // rows_per_blk,)
    return pl.pallas_call(
        functools.partial(_mm_body, split=split),
        grid=grid,
        in_specs=[
            pl.BlockSpec((rows_per_blk, k), lambda i: (i, 0)),
            pl.BlockSpec((k, m), lambda i: (0, 0)),
        ],
        out_specs=[
            pl.BlockSpec((rows_per_blk, split), lambda i: (i, 0)),
            pl.BlockSpec((rows_per_blk, m - split), lambda i: (i, 0)),
        ],
        out_shape=[
            jax.ShapeDtypeStruct((n, split), jnp.bfloat16),
            jax.ShapeDtypeStruct((n, m - split), jnp.float32),
        ],
    )(x, w)


def _mid_body(p_ref, w_ref, b_ref, t2_ref, er_ref):
    a = p_ref[0] + p_ref[1]
    d = a[:, D1:D1 + H1]
    r = 1.0 / jnp.where(d != 0.0, d, 1.0)
    pieces = []
    for h in range(H1):
        pieces.append(a[:, F1 * h:F1 * h + F1] * r[:, h:h + 1])
    o = jnp.concatenate(pieces, axis=1)
    hb = jnp.maximum(o + b_ref[...], 0.0)
    y = jnp.dot(hb, w_ref[...], preferred_element_type=jnp.float32)
    t2_ref[...] = y[:, :GW2].astype(jnp.bfloat16)
    er_ref[...] = y[:, GW2:]


def _tc_mid(p1, w, b1row, rows_per_blk=2000):
    grid = (N // rows_per_blk,)
    return pl.pallas_call(
        _mid_body,
        grid=grid,
        in_specs=[
            pl.BlockSpec((2, rows_per_blk, TW1), lambda i: (0, i, 0)),
            pl.BlockSpec(w.shape, lambda i: (0, 0)),
            pl.BlockSpec((1, D1), lambda i: (0, 0)),
        ],
        out_specs=[
            pl.BlockSpec((rows_per_blk, GW2), lambda i: (i, 0)),
            pl.BlockSpec((rows_per_blk, ERW), lambda i: (i, 0)),
        ],
        out_shape=[
            jax.ShapeDtypeStruct((N, GW2), jnp.bfloat16),
            jax.ShapeDtypeStruct((N, ERW), jnp.float32),
        ],
    )(p1, w, b1row)


def _final_body(p_ref, b_ref, o_ref):
    a = p_ref[0] + p_ref[1]
    d = a[:, F2:F2 + 1]
    d = jnp.where(d != 0.0, d, 1.0)
    z = a[:, :F2] / d + b_ref[...]
    m = jnp.max(z, axis=1, keepdims=True)
    lse = m + jnp.log(jnp.sum(jnp.exp(z - m), axis=1, keepdims=True))
    o_ref[...] = z - lse


def _tc_final(p2, b2row, rows_per_blk=2000):
    grid = (N // rows_per_blk,)
    return pl.pallas_call(
        _final_body,
        grid=grid,
        in_specs=[
            pl.BlockSpec((2, rows_per_blk, TW2), lambda i: (0, i, 0)),
            pl.BlockSpec((1, F2), lambda i: (0, 0)),
        ],
        out_specs=pl.BlockSpec((rows_per_blk, F2), lambda i: (i, 0)),
        out_shape=jax.ShapeDtypeStruct((N, F2), jnp.float32),
    )(p2, b2row)


def _sc_layer1_body(t1_hbm, er_hbm, src_hbm, dst_hbm, z_hbm, out_hbm,
                    srcv, dstv, t1b, erb, msgb, acc,
                    gsem1, gsem2, ssem):
    cid = lax.axis_index("c")
    sid = lax.axis_index("s")
    wid = cid * 16 + sid
    _rowcopy(z_hbm, acc, sid)
    pltpu.sync_copy(src_hbm.at[wid], srcv)
    pltpu.sync_copy(dst_hbm.at[wid], dstv)
    plsc.subcore_barrier()

    il = lax.iota(jnp.int32, 16)
    head_sel = il >> 3
    takeidx = [2 * q + head_sel for q in range(4)]

    def gathers(c, b):
        pltpu.async_copy(t1_hbm.at[srcv.at[c]], t1b.at[b], gsem1.at[b])
        pltpu.async_copy(er_hbm.at[dstv.at[c]], erb.at[b], gsem2.at[b])

    for b in range(NBUF):
        gathers(b, b)

    @pl.loop(0, NCH, step=NBUF)
    def _chunk(j):
        for b in range(NBUF):
            c = j + b
            pltpu.make_async_copy(t1_hbm.at[srcv.at[c]],
                                  t1b.at[b], gsem1.at[b]).wait()
            pltpu.make_async_copy(er_hbm.at[dstv.at[c]],
                                  erb.at[b], gsem2.at[b]).wait()

            @pl.when(c >= NBUF)
            def _():
                pltpu.make_async_copy(msgb.at[b], acc.at[dstv.at[c]],
                                      ssem.at[b]).wait()

            @plsc.parallel_loop(0, CHUNK, unroll=8)
            def _edge(e):
                elv, _ = _unpack32(t1b[b, e, pl.ds(D1, 32)])
                erv = erb[b, e, :]
                ev = elv + erv
                ev = jnp.where(ev >= 0.0, ev, 0.2 * ev)
                sv = jnp.exp(ev)
                msgb[b, e, pl.ds(D1, 16)] = sv
                for q2 in range(2):
                    ha, hbv = _unpack32(t1b[b, e, pl.ds(32 * q2, 32)])
                    msgb[b, e, pl.ds(32 * q2, 16)] = (
                        ha * _vgather(sv, takeidx[2 * q2]))
                    msgb[b, e, pl.ds(32 * q2 + 16, 16)] = (
                        hbv * _vgather(sv, takeidx[2 * q2 + 1]))

            pltpu.async_copy(msgb.at[b], acc.at[dstv.at[c]], ssem.at[b],
                             add=True)

            @pl.when(c + NBUF < NCH)
            def _():
                gathers(c + NBUF, b)

    for b in range(NBUF):
        pltpu.make_async_copy(msgb.at[b], acc.at[pl.ds(0, CHUNK)],
                              ssem.at[b]).wait()

    plsc.subcore_barrier()
    _rowcopy(acc, out_hbm.at[cid], sid)


def _rowcopy(src, dst, sid):
    @pl.when(sid < 15)
    def _():
        st = pl.multiple_of(sid * R0, 8)
        pltpu.sync_copy(src.at[pl.ds(st, R0)], dst.at[pl.ds(st, R0)])

    @pl.when(sid == 15)
    def _():
        pltpu.sync_copy(src.at[pl.ds(15 * R0, RLAST)],
                        dst.at[pl.ds(15 * R0, RLAST)])


def _sc_layer1(t1, er1, src_r, dst_r, z80):
    k = pl.kernel(
        _sc_layer1_body,
        out_type=jax.ShapeDtypeStruct((2, N, TW1), jnp.float32),
        mesh=_mesh(),
        compiler_params=_sc_params(),
        scratch_types=[
            pltpu.VMEM((NCH, CHUNK), jnp.int32),
            pltpu.VMEM((NCH, CHUNK), jnp.int32),
            pltpu.VMEM((NBUF, CHUNK, GW1), jnp.bfloat16),
            pltpu.VMEM((NBUF, CHUNK, ERW), jnp.float32),
            pltpu.VMEM((NBUF, CHUNK, TW1), jnp.float32),
            pltpu.VMEM_SHARED((N, TW1), jnp.float32),
            pltpu.SemaphoreType.DMA((NBUF,)),
            pltpu.SemaphoreType.DMA((NBUF,)),
            pltpu.SemaphoreType.DMA((NBUF,)),
        ],
    )
    return k(t1, er1, src_r, dst_r, z80)


def _sc_layer2_body(t2_hbm, er_hbm, src_hbm, dst_hbm, z_hbm, out_hbm,
                    srcv, dstv, t2b, erb, msgb, acc,
                    gsem1, gsem2, ssem):
    cid = lax.axis_index("c")
    sid = lax.axis_index("s")
    wid = cid * 16 + sid
    _rowcopy(z_hbm, acc, sid)
    pltpu.sync_copy(src_hbm.at[wid], srcv)
    pltpu.sync_copy(dst_hbm.at[wid], dstv)
    plsc.subcore_barrier()

    il = lax.iota(jnp.int32, 16)
    zero = jnp.zeros((16,), jnp.float32)
    full8 = jnp.full((16,), 8, jnp.int32)
    full0 = jnp.zeros((16,), jnp.int32)

    def gathers(c, b):
        pltpu.async_copy(t2_hbm.at[srcv.at[c]], t2b.at[b], gsem1.at[b])
        pltpu.async_copy(er_hbm.at[dstv.at[c]], erb.at[b], gsem2.at[b])

    for b in range(NBUF):
        gathers(b, b)

    @pl.loop(0, NCH, step=NBUF)
    def _chunk(j):
        for b in range(NBUF):
            c = j + b
            pltpu.make_async_copy(t2_hbm.at[srcv.at[c]],
                                  t2b.at[b], gsem1.at[b]).wait()
            pltpu.make_async_copy(er_hbm.at[dstv.at[c]],
                                  erb.at[b], gsem2.at[b]).wait()

            @pl.when(c >= NBUF)
            def _():
                pltpu.make_async_copy(msgb.at[b], acc.at[dstv.at[c]],
                                      ssem.at[b]).wait()

            @plsc.parallel_loop(0, CHUNK, unroll=8)
            def _edge(e):
                erv = erb[b, e, :]
                m0, m1 = _unpack32(t2b[b, e, pl.ds(0, 32)])
                m2, _ = _unpack32(t2b[b, e, pl.ds(32, 32)])
                ev = _vgather(m2, full8) + _vgather(erv, full0)
                ev = jnp.where(ev >= 0.0, ev, 0.2 * ev)
                sb = jnp.exp(ev)
                msgb[b, e, pl.ds(0, 16)] = m0 * sb
                msgb[b, e, pl.ds(16, 16)] = m1 * sb
                mv = jnp.where(il < 8, m2 * sb,
                               jnp.where(il == 8, sb, zero))
                msgb[b, e, pl.ds(32, 16)] = mv

            pltpu.async_copy(msgb.at[b], acc.at[dstv.at[c]], ssem.at[b],
                             add=True)

            @pl.when(c + NBUF < NCH)
            def _():
                gathers(c + NBUF, b)

    for b in range(NBUF):
        pltpu.make_async_copy(msgb.at[b], acc.at[pl.ds(0, CHUNK)],
                              ssem.at[b]).wait()

    plsc.subcore_barrier()
    _rowcopy(acc, out_hbm.at[cid], sid)


def _sc_layer2(t2, er2, src_r, dst_r, z48):
    k = pl.kernel(
        _sc_layer2_body,
        out_type=jax.ShapeDtypeStruct((2, N, TW2), jnp.float32),
        mesh=_mesh(),
        compiler_params=_sc_params(),
        scratch_types=[
            pltpu.VMEM((NCH, CHUNK), jnp.int32),
            pltpu.VMEM((NCH, CHUNK), jnp.int32),
            pltpu.VMEM((NBUF, CHUNK, GW2), jnp.bfloat16),
            pltpu.VMEM((NBUF, CHUNK, ERW), jnp.float32),
            pltpu.VMEM((NBUF, CHUNK, TW2), jnp.float32),
            pltpu.VMEM_SHARED((N, TW2), jnp.float32),
            pltpu.SemaphoreType.DMA((NBUF,)),
            pltpu.SemaphoreType.DMA((NBUF,)),
            pltpu.SemaphoreType.DMA((NBUF,)),
        ],
    )
    return k(t2, er2, src_r, dst_r, z48)


def kernel(x, edge_index, W1, al1, ar1, b1, W2, al2, ar2, b2):
    w1r = W1.reshape(H1, F1, F_IN)
    a_l1 = jnp.einsum("hfk,hf->kh", w1r, al1[0])
    a_r1 = jnp.einsum("hfk,hf->kh", w1r, ar1[0])
    t1_nat = jnp.concatenate(
        [W1.T, a_l1, jnp.zeros((F_IN, 24), jnp.float32)], axis=1)
    wc1 = jnp.concatenate(
        [t1_nat[:, _bf16_perm(GW1)], a_r1,
         jnp.zeros((F_IN, 8), jnp.float32)], axis=1)

    w2r = W2.reshape(H2, F2, D1)
    a_l2 = jnp.einsum("hfk,hf->kh", w2r, al2[0])
    a_r2 = jnp.einsum("hfk,hf->kh", w2r, ar2[0])
    t2_nat = jnp.concatenate(
        [W2.T, a_l2, jnp.zeros((D1, 23), jnp.float32)], axis=1)
    wc2 = jnp.concatenate(
        [t2_nat[:, _bf16_perm(GW2)], a_r2,
         jnp.zeros((D1, 15), jnp.float32)], axis=1)

    src_r = edge_index[0].reshape(NTILES, NCH, CHUNK)
    dst_r = edge_index[1].reshape(NTILES, NCH, CHUNK)
    z80 = jnp.zeros((N, TW1), jnp.float32)
    z48 = jnp.zeros((N, TW2), jnp.float32)
    b1row = b1.reshape(1, D1)
    b2row = b2.reshape(1, F2)

    t1, er1 = _tc_project(x, wc1, GW1)
    p1 = _sc_layer1(t1, er1, src_r, dst_r, z80)
    t2, er2 = _tc_mid(p1, wc2, b1row)
    p2 = _sc_layer2(t2, er2, src_r, dst_r, z48)
    return _tc_final(p2, b2row)


# speedup vs baseline: 201.7786x; 1.0166x over previous
import dataclasses
import functools

import jax
import jax.numpy as jnp
from jax import lax
from jax.experimental import pallas as pl
from jax.experimental.pallas import tpu as pltpu
from jax.experimental.pallas import tpu_sc as plsc

N = 10000
E = 320000
F_IN = 128
H1, F1 = 8, 8
D1 = H1 * F1
H2, F2 = 1, 40
D2 = H2 * F2

TW1 = 72
TW2 = 48
GW1 = 96
GW2 = 64
ERW = 16

NTILES = 32
CHUNK = 100
EPT = E // NTILES
NCH = EPT // CHUNK
NBUF = 4
R0 = 624
RLAST = N - 15 * R0

def _bf16_perm(width):
    p = []
    for blk in range(width // 32):
        for i in range(16):
            p.append(32 * blk + i)
            p.append(32 * blk + 16 + i)
    return p


def _unpack32(v):
    return plsc.unpack(v, format=plsc.PackFormat.INTERLEAVED,
                       preferred_element_type=jnp.float32)


def _vgather(x, idx):
    dnums = lax.GatherDimensionNumbers(
        offset_dims=(), collapsed_slice_dims=(0,), start_index_map=(0,))
    return lax.gather(x, idx[:, None], dnums, (1,),
                      mode=lax.GatherScatterMode.PROMISE_IN_BOUNDS)


def _mesh():
    return plsc.VectorSubcoreMesh(core_axis_name="c", subcore_axis_name="s")


def _sc_params():
    cp = pltpu.CompilerParams()
    fields = pltpu.CompilerParams.__dataclass_fields__
    if "needs_layout_passes" in fields:
        cp = dataclasses.replace(cp, needs_layout_passes=False)
    if "use_tc_tiling_on_sc" in fields:
        cp = dataclasses.replace(cp, use_tc_tiling_on_sc=False)
    return cp


def _mm_body(x_ref, w_ref, a_ref, b_ref, split):
    y = jnp.dot(x_ref[...], w_ref[...], preferred_element_type=jnp.float32)
    a_ref[...] = y[:, :split].astype(a_ref.dtype)
    b_ref[...] = y[:, split:]


def _tc_project(x, w, split, rows_per_blk=2000):
    n, k = x.shape
    m = w.shape[1]
    grid = (n // rows_per_blk,)
    return pl.pallas_call(
        functools.partial(_mm_body, split=split),
        grid=grid,
        in_specs=[
            pl.BlockSpec((rows_per_blk, k), lambda i: (i, 0)),
            pl.BlockSpec((k, m), lambda i: (0, 0)),
        ],
        out_specs=[
            pl.BlockSpec((rows_per_blk, split), lambda i: (i, 0)),
            pl.BlockSpec((rows_per_blk, m - split), lambda i: (i, 0)),
        ],
        out_shape=[
            jax.ShapeDtypeStruct((n, split), jnp.bfloat16),
            jax.ShapeDtypeStruct((n, m - split), jnp.float32),
        ],
    )(x, w)


def _mid_body(p_ref, w_ref, b_ref, t2_ref, er_ref):
    a = p_ref[0] + p_ref[1]
    d = a[:, D1:D1 + H1]
    r = 1.0 / jnp.where(d != 0.0, d, 1.0)
    pieces = []
    for h in range(H1):
        pieces.append(a[:, F1 * h:F1 * h + F1] * r[:, h:h + 1])
    o = jnp.concatenate(pieces, axis=1)
    hb = jnp.maximum(o + b_ref[...], 0.0)
    y = jnp.dot(hb, w_ref[...], preferred_element_type=jnp.float32)
    t2_ref[...] = y[:, :GW2].astype(jnp.bfloat16)
    er_ref[...] = y[:, GW2:]


def _tc_mid(p1, w, b1row, rows_per_blk=2000):
    grid = (N // rows_per_blk,)
    return pl.pallas_call(
        _mid_body,
        grid=grid,
        in_specs=[
            pl.BlockSpec((2, rows_per_blk, TW1), lambda i: (0, i, 0)),
            pl.BlockSpec(w.shape, lambda i: (0, 0)),
            pl.BlockSpec((1, D1), lambda i: (0, 0)),
        ],
        out_specs=[
            pl.BlockSpec((rows_per_blk, GW2), lambda i: (i, 0)),
            pl.BlockSpec((rows_per_blk, ERW), lambda i: (i, 0)),
        ],
        out_shape=[
            jax.ShapeDtypeStruct((N, GW2), jnp.bfloat16),
            jax.ShapeDtypeStruct((N, ERW), jnp.float32),
        ],
    )(p1, w, b1row)


def _final_body(p_ref, b_ref, o_ref):
    a = p_ref[0] + p_ref[1]
    d = a[:, F2:F2 + 1]
    d = jnp.where(d != 0.0, d, 1.0)
    z = a[:, :F2] / d + b_ref[...]
    m = jnp.max(z, axis=1, keepdims=True)
    lse = m + jnp.log(jnp.sum(jnp.exp(z - m), axis=1, keepdims=True))
    o_ref[...] = z - lse


def _tc_final(p2, b2row, rows_per_blk=2000):
    grid = (N // rows_per_blk,)
    return pl.pallas_call(
        _final_body,
        grid=grid,
        in_specs=[
            pl.BlockSpec((2, rows_per_blk, TW2), lambda i: (0, i, 0)),
            pl.BlockSpec((1, F2), lambda i: (0, 0)),
        ],
        out_specs=pl.BlockSpec((rows_per_blk, F2), lambda i: (i, 0)),
        out_shape=jax.ShapeDtypeStruct((N, F2), jnp.float32),
    )(p2, b2row)


def _sc_layer1_body(t1_hbm, er_hbm, src_hbm, dst_hbm, z_hbm, out_hbm,
                    srcv, dstv, t1b, erb, msgb, acc,
                    gsem1, gsem2, ssem):
    cid = lax.axis_index("c")
    sid = lax.axis_index("s")
    wid = cid * 16 + sid
    _rowcopy(z_hbm, acc, sid)
    pltpu.sync_copy(src_hbm.at[wid], srcv)
    pltpu.sync_copy(dst_hbm.at[wid], dstv)
    plsc.subcore_barrier()

    il = lax.iota(jnp.int32, 16)
    head_sel = il >> 3
    takeidx = [2 * q + head_sel for q in range(4)]

    def gathers(c, b):
        pltpu.async_copy(t1_hbm.at[srcv.at[c]], t1b.at[b], gsem1.at[b])
        pltpu.async_copy(er_hbm.at[dstv.at[c]], erb.at[b], gsem2.at[b])

    for b in range(NBUF):
        gathers(b, b)

    @pl.loop(0, NCH, step=NBUF)
    def _chunk(j):
        for b in range(NBUF):
            c = j + b
            pltpu.make_async_copy(t1_hbm.at[srcv.at[c]],
                                  t1b.at[b], gsem1.at[b]).wait()
            pltpu.make_async_copy(er_hbm.at[dstv.at[c]],
                                  erb.at[b], gsem2.at[b]).wait()

            @pl.when(c >= NBUF)
            def _():
                pltpu.make_async_copy(msgb.at[b], acc.at[dstv.at[c]],
                                      ssem.at[b]).wait()

            @plsc.parallel_loop(0, CHUNK, unroll=4)
            def _edge(e):
                elv, _ = _unpack32(t1b[b, e, pl.ds(D1, 32)])
                erv = erb[b, e, :]
                ev = elv + erv
                ev = jnp.where(ev >= 0.0, ev, 0.2 * ev)
                sv = jnp.exp(ev)
                plsc.store_scatter(msgb.at[b],
                                   (jnp.full((16,), e, jnp.int32), D1 + il),
                                   sv, mask=il < 8)
                for q2 in range(2):
                    ha, hbv = _unpack32(t1b[b, e, pl.ds(32 * q2, 32)])
                    msgb[b, e, pl.ds(32 * q2, 16)] = (
                        ha * _vgather(sv, takeidx[2 * q2]))
                    msgb[b, e, pl.ds(32 * q2 + 16, 16)] = (
                        hbv * _vgather(sv, takeidx[2 * q2 + 1]))

            pltpu.async_copy(msgb.at[b], acc.at[dstv.at[c]], ssem.at[b],
                             add=True)

            @pl.when(c + NBUF < NCH)
            def _():
                gathers(c + NBUF, b)

    for b in range(NBUF):
        pltpu.make_async_copy(msgb.at[b], acc.at[pl.ds(0, CHUNK)],
                              ssem.at[b]).wait()

    plsc.subcore_barrier()
    _rowcopy(acc, out_hbm.at[cid], sid)


def _rowcopy(src, dst, sid):
    @pl.when(sid < 15)
    def _():
        st = pl.multiple_of(sid * R0, 8)
        pltpu.sync_copy(src.at[pl.ds(st, R0)], dst.at[pl.ds(st, R0)])

    @pl.when(sid == 15)
    def _():
        pltpu.sync_copy(src.at[pl.ds(15 * R0, RLAST)],
                        dst.at[pl.ds(15 * R0, RLAST)])


def _sc_layer1(t1, er1, src_r, dst_r, z80):
    k = pl.kernel(
        _sc_layer1_body,
        out_type=jax.ShapeDtypeStruct((2, N, TW1), jnp.float32),
        mesh=_mesh(),
        compiler_params=_sc_params(),
        scratch_types=[
            pltpu.VMEM((NCH, CHUNK), jnp.int32),
            pltpu.VMEM((NCH, CHUNK), jnp.int32),
            pltpu.VMEM((NBUF, CHUNK, GW1), jnp.bfloat16),
            pltpu.VMEM((NBUF, CHUNK, ERW), jnp.float32),
            pltpu.VMEM((NBUF, CHUNK, TW1), jnp.float32),
            pltpu.VMEM_SHARED((N, TW1), jnp.float32),
            pltpu.SemaphoreType.DMA((NBUF,)),
            pltpu.SemaphoreType.DMA((NBUF,)),
            pltpu.SemaphoreType.DMA((NBUF,)),
        ],
    )
    return k(t1, er1, src_r, dst_r, z80)


def _sc_layer2_body(t2_hbm, er_hbm, src_hbm, dst_hbm, z_hbm, out_hbm,
                    srcv, dstv, t2b, erb, msgb, acc,
                    gsem1, gsem2, ssem):
    cid = lax.axis_index("c")
    sid = lax.axis_index("s")
    wid = cid * 16 + sid
    _rowcopy(z_hbm, acc, sid)
    pltpu.sync_copy(src_hbm.at[wid], srcv)
    pltpu.sync_copy(dst_hbm.at[wid], dstv)
    plsc.subcore_barrier()

    il = lax.iota(jnp.int32, 16)
    zero = jnp.zeros((16,), jnp.float32)
    full8 = jnp.full((16,), 8, jnp.int32)
    full0 = jnp.zeros((16,), jnp.int32)

    def gathers(c, b):
        pltpu.async_copy(t2_hbm.at[srcv.at[c]], t2b.at[b], gsem1.at[b])
        pltpu.async_copy(er_hbm.at[dstv.at[c]], erb.at[b], gsem2.at[b])

    for b in range(NBUF):
        gathers(b, b)

    @pl.loop(0, NCH, step=NBUF)
    def _chunk(j):
        for b in range(NBUF):
            c = j + b
            pltpu.make_async_copy(t2_hbm.at[srcv.at[c]],
                                  t2b.at[b], gsem1.at[b]).wait()
            pltpu.make_async_copy(er_hbm.at[dstv.at[c]],
                                  erb.at[b], gsem2.at[b]).wait()

            @pl.when(c >= NBUF)
            def _():
                pltpu.make_async_copy(msgb.at[b], acc.at[dstv.at[c]],
                                      ssem.at[b]).wait()

            @plsc.parallel_loop(0, CHUNK, unroll=4)
            def _edge(e):
                erv = erb[b, e, :]
                m0, m1 = _unpack32(t2b[b, e, pl.ds(0, 32)])
                m2, _ = _unpack32(t2b[b, e, pl.ds(32, 32)])
                ev = _vgather(m2, full8) + _vgather(erv, full0)
                ev = jnp.where(ev >= 0.0, ev, 0.2 * ev)
                sb = jnp.exp(ev)
                msgb[b, e, pl.ds(0, 16)] = m0 * sb
                msgb[b, e, pl.ds(16, 16)] = m1 * sb
                mv = jnp.where(il < 8, m2 * sb,
                               jnp.where(il == 8, sb, zero))
                msgb[b, e, pl.ds(32, 16)] = mv

            pltpu.async_copy(msgb.at[b], acc.at[dstv.at[c]], ssem.at[b],
                             add=True)

            @pl.when(c + NBUF < NCH)
            def _():
                gathers(c + NBUF, b)

    for b in range(NBUF):
        pltpu.make_async_copy(msgb.at[b], acc.at[pl.ds(0, CHUNK)],
                              ssem.at[b]).wait()

    plsc.subcore_barrier()
    _rowcopy(acc, out_hbm.at[cid], sid)


def _sc_layer2(t2, er2, src_r, dst_r, z48):
    k = pl.kernel(
        _sc_layer2_body,
        out_type=jax.ShapeDtypeStruct((2, N, TW2), jnp.float32),
        mesh=_mesh(),
        compiler_params=_sc_params(),
        scratch_types=[
            pltpu.VMEM((NCH, CHUNK), jnp.int32),
            pltpu.VMEM((NCH, CHUNK), jnp.int32),
            pltpu.VMEM((NBUF, CHUNK, GW2), jnp.bfloat16),
            pltpu.VMEM((NBUF, CHUNK, ERW), jnp.float32),
            pltpu.VMEM((NBUF, CHUNK, TW2), jnp.float32),
            pltpu.VMEM_SHARED((N, TW2), jnp.float32),
            pltpu.SemaphoreType.DMA((NBUF,)),
            pltpu.SemaphoreType.DMA((NBUF,)),
            pltpu.SemaphoreType.DMA((NBUF,)),
        ],
    )
    return k(t2, er2, src_r, dst_r, z48)


def kernel(x, edge_index, W1, al1, ar1, b1, W2, al2, ar2, b2):
    w1r = W1.reshape(H1, F1, F_IN)
    a_l1 = jnp.einsum("hfk,hf->kh", w1r, al1[0])
    a_r1 = jnp.einsum("hfk,hf->kh", w1r, ar1[0])
    t1_nat = jnp.concatenate(
        [W1.T, a_l1, jnp.zeros((F_IN, 24), jnp.float32)], axis=1)
    wc1 = jnp.concatenate(
        [t1_nat[:, _bf16_perm(GW1)], a_r1,
         jnp.zeros((F_IN, 8), jnp.float32)], axis=1)

    w2r = W2.reshape(H2, F2, D1)
    a_l2 = jnp.einsum("hfk,hf->kh", w2r, al2[0])
    a_r2 = jnp.einsum("hfk,hf->kh", w2r, ar2[0])
    t2_nat = jnp.concatenate(
        [W2.T, a_l2, jnp.zeros((D1, 23), jnp.float32)], axis=1)
    wc2 = jnp.concatenate(
        [t2_nat[:, _bf16_perm(GW2)], a_r2,
         jnp.zeros((D1, 15), jnp.float32)], axis=1)

    src_r = edge_index[0].reshape(NTILES, NCH, CHUNK)
    dst_r = edge_index[1].reshape(NTILES, NCH, CHUNK)
    z80 = jnp.zeros((N, TW1), jnp.float32)
    z48 = jnp.zeros((N, TW2), jnp.float32)
    b1row = b1.reshape(1, D1)
    b2row = b2.reshape(1, F2)

    t1, er1 = _tc_project(x, wc1, GW1)
    p1 = _sc_layer1(t1, er1, src_r, dst_r, z80)
    t2, er2 = _tc_mid(p1, wc2, b1row)
    p2 = _sc_layer2(t2, er2, src_r, dst_r, z48)
    return _tc_final(p2, b2row)
